# baseline (speedup 1.0000x reference)
import functools

import jax
import jax.numpy as jnp
from jax import lax
from jax.experimental import pallas as pl
from jax.experimental.pallas import tpu as pltpu
from jax.experimental.pallas import tpu_sc as plsc

N = 10000
E = 320000
IN_CH = 128
HID = 16
HEADS = 8
OUT_CH = 64

_HIGH = jax.lax.Precision.HIGHEST
_BM = 1000
_G = N // _BM
CH = 128
NFULL = N // CH
NTAIL = N - NFULL * CH
NCPT = 159
E_PAD = 16 * NCPT * CH
NROWS = N + 8


def _sc_mesh():
    return plsc.VectorSubcoreMesh(core_axis_name="c", subcore_axis_name="s")


def _dot(a, b):
    return jnp.dot(a, b, precision=_HIGH, preferred_element_type=jnp.float32)


def _tc1_body(x_ref, w_ref, as_ref, ad_ref, h_ref, q_ref, r_ref, qm_ref, rm_ref):
    i = pl.program_id(0)
    h = _dot(x_ref[...], w_ref[...])
    h_ref[0] = h[:, :IN_CH // 2]
    h_ref[1] = h[:, IN_CH // 2:]
    q = _dot(h, as_ref[...])
    r = _dot(h, ad_ref[...])
    q_ref[...] = q
    r_ref[...] = r
    qm = jnp.max(q, axis=0, keepdims=True)
    rm = jnp.max(r, axis=0, keepdims=True)

    @pl.when(i == 0)
    def _():
        qm_ref[...] = qm
        rm_ref[...] = rm

    @pl.when(i > 0)
    def _():
        qm_ref[...] = jnp.maximum(qm_ref[...], qm)
        rm_ref[...] = jnp.maximum(rm_ref[...], rm)


def _tc2_body(acc_ref, den_ref, b1_ref, exp_ref, w2_ref, as_ref, ad_ref,
              h2_ref, s_ref, d_ref, sm_ref, dm_ref):
    i = pl.program_id(0)
    acc = jnp.concatenate([acc_ref[0], acc_ref[1]], axis=1)
    den8 = den_ref[0][:, :8] + den_ref[1][:, :8]
    den128 = _dot(den8, exp_ref[...])
    h1 = acc / (den128 + 1e-16) + b1_ref[...]
    h2 = _dot(h1, w2_ref[...])
    h2_ref[0] = h2[:, :OUT_CH // 2]
    h2_ref[1] = h2[:, OUT_CH // 2:]
    s = _dot(h2, as_ref[...])
    d = _dot(h2, ad_ref[...])
    s_ref[...] = s
    d_ref[...] = d
    sm = jnp.max(s, axis=0, keepdims=True)
    dm = jnp.max(d, axis=0, keepdims=True)

    @pl.when(i == 0)
    def _():
        sm_ref[...] = sm
        dm_ref[...] = dm

    @pl.when(i > 0)
    def _():
        sm_ref[...] = jnp.maximum(sm_ref[...], sm)
        dm_ref[...] = jnp.maximum(dm_ref[...], dm)


def _tc3_body(acc_ref, den_ref, b2_ref, out_ref):
    acc = jnp.concatenate([acc_ref[0], acc_ref[1]], axis=1)
    den = den_ref[0][:, 0:1] + den_ref[1][:, 0:1]
    o = acc / (den + 1e-16) + b2_ref[...]
    m = jnp.max(o, axis=1, keepdims=True)
    z = o - m
    lse = jnp.log(jnp.sum(jnp.exp(z), axis=1, keepdims=True))
    out_ref[...] = z - lse


def _tc1(x, W1, As1, Ad1):
    return pl.pallas_call(
        _tc1_body,
        grid=(_G,),
        in_specs=[
            pl.BlockSpec((_BM, IN_CH), lambda i: (i, 0)),
            pl.BlockSpec((IN_CH, IN_CH), lambda i: (0, 0)),
            pl.BlockSpec((IN_CH, 16), lambda i: (0, 0)),
            pl.BlockSpec((IN_CH, 16), lambda i: (0, 0)),
        ],
        out_specs=[
            pl.BlockSpec((2, _BM, IN_CH // 2), lambda i: (0, i, 0)),
            pl.BlockSpec((_BM, 16), lambda i: (i, 0)),
            pl.BlockSpec((_BM, 16), lambda i: (i, 0)),
            pl.BlockSpec((1, 16), lambda i: (0, 0)),
            pl.BlockSpec((1, 16), lambda i: (0, 0)),
        ],
        out_shape=[
            jax.ShapeDtypeStruct((2, N, IN_CH // 2), jnp.float32),
            jax.ShapeDtypeStruct((N, 16), jnp.float32),
            jax.ShapeDtypeStruct((N, 16), jnp.float32),
            jax.ShapeDtypeStruct((1, 16), jnp.float32),
            jax.ShapeDtypeStruct((1, 16), jnp.float32),
        ],
    )(x, W1, As1, Ad1)


def _tc2(acc1, den1, b1, Exp8, W2, As2, Ad2):
    return pl.pallas_call(
        _tc2_body,
        grid=(_G,),
        in_specs=[
            pl.BlockSpec((2, _BM, IN_CH // 2), lambda i: (0, i, 0)),
            pl.BlockSpec((2, _BM, 16), lambda i: (0, i, 0)),
            pl.BlockSpec((1, IN_CH), lambda i: (0, 0)),
            pl.BlockSpec((8, IN_CH), lambda i: (0, 0)),
            pl.BlockSpec((IN_CH, OUT_CH), lambda i: (0, 0)),
            pl.BlockSpec((OUT_CH, 16), lambda i: (0, 0)),
            pl.BlockSpec((OUT_CH, 16), lambda i: (0, 0)),
        ],
        out_specs=[
            pl.BlockSpec((2, _BM, OUT_CH // 2), lambda i: (0, i, 0)),
            pl.BlockSpec((_BM, 16), lambda i: (i, 0)),
            pl.BlockSpec((_BM, 16), lambda i: (i, 0)),
            pl.BlockSpec((1, 16), lambda i: (0, 0)),
            pl.BlockSpec((1, 16), lambda i: (0, 0)),
        ],
        out_shape=[
            jax.ShapeDtypeStruct((2, N, OUT_CH // 2), jnp.float32),
            jax.ShapeDtypeStruct((N, 16), jnp.float32),
            jax.ShapeDtypeStruct((N, 16), jnp.float32),
            jax.ShapeDtypeStruct((1, 16), jnp.float32),
            jax.ShapeDtypeStruct((1, 16), jnp.float32),
        ],
    )(acc1, den1, b1, Exp8, W2, As2, Ad2)


def _tc3(acc2, den2, b2):
    return pl.pallas_call(
        _tc3_body,
        grid=(_G,),
        in_specs=[
            pl.BlockSpec((2, _BM, OUT_CH // 2), lambda i: (0, i, 0)),
            pl.BlockSpec((2, _BM, 16), lambda i: (0, i, 0)),
            pl.BlockSpec((1, OUT_CH), lambda i: (0, 0)),
        ],
        out_specs=pl.BlockSpec((_BM, OUT_CH), lambda i: (i, 0)),
        out_shape=jax.ShapeDtypeStruct((N, OUT_CH), jnp.float32),
    )(acc2, den2, b2)


def _sc_edge_pass(h2s, Q2s, R2s, C2s, src2d, dst2d, F):
    FH = F // 2
    ngrp = FH // 16

    @functools.partial(
        pl.kernel,
        out_type=[
            jax.ShapeDtypeStruct((2, N, FH), jnp.float32),
            jax.ShapeDtypeStruct((2, N, 16), jnp.float32),
        ],
        mesh=_sc_mesh(),
        compiler_params=pltpu.CompilerParams(use_tc_tiling_on_sc=False),
        scratch_types=[
            pltpu.VMEM((NCPT, CH), jnp.int32),
            [pltpu.VMEM((CH,), jnp.int32)] * 3,
            [pltpu.VMEM((CH, 16), jnp.float32)] * 3,
            [pltpu.VMEM((CH, 16), jnp.float32)] * 3,
            [pltpu.VMEM((CH, FH), jnp.float32)] * 3,
            [pltpu.VMEM((CH, 16), jnp.float32)] * 3,
            pltpu.VMEM((1, 16), jnp.float32),
            pltpu.VMEM_SHARED((NROWS, FH), jnp.float32),
            pltpu.VMEM_SHARED((NROWS, 16), jnp.float32),
            [pltpu.SemaphoreType.DMA] * 3,
            [pltpu.SemaphoreType.DMA] * 3,
            [pltpu.SemaphoreType.DMA] * 3,
        ],
    )
    def k(h_hbm, q_hbm, r_hbm, c_hbm, s_hbm, d_hbm, acc_out, den_out,
          dslab, SB, QS, RD, HS, EB, cvec, acc_sp, den_sp, IS, GS, SS):
        cid = lax.axis_index("c")
        sid = lax.axis_index("s")

        hs0, eb0 = HS[0], EB[0]

        @pl.loop(0, CH)
        def _(r2):
            for j in range(ngrp):
                hs0[r2, pl.ds(j * 16, 16)] = jnp.zeros((16,), jnp.float32)
            eb0[r2, :] = jnp.zeros((16,), jnp.float32)

        for j in range(5):
            i = j * 16 + sid

            @pl.when(i < NFULL)
            def _():
                pltpu.sync_copy(hs0, acc_sp.at[pl.ds(i * CH, CH)])
                pltpu.sync_copy(eb0, den_sp.at[pl.ds(i * CH, CH)])

        @pl.when(sid == 15)
        def _():
            pltpu.sync_copy(hs0.at[pl.ds(0, NTAIL)],
                            acc_sp.at[pl.ds(NFULL * CH, NTAIL)])
            pltpu.sync_copy(eb0.at[pl.ds(0, NTAIL)],
                            den_sp.at[pl.ds(NFULL * CH, NTAIL)])

        pltpu.sync_copy(c_hbm.at[pl.ds(cid, 1)], cvec)
        pltpu.sync_copy(d_hbm.at[pl.ds(sid * NCPT, NCPT)], dslab)
        plsc.subcore_barrier()
        cv = cvec[0, :]
        base = sid * NCPT

        def fire_sidx(c, b):
            pltpu.async_copy(s_hbm.at[base + c], SB[b], IS[b])

        def wait_sidx(b):
            pltpu.make_async_copy(s_hbm.at[base], SB[b], IS[b]).wait()

        def fire_g(c, b):
            pltpu.async_copy(q_hbm.at[cid].at[SB[b]], QS[b], GS[b])
            pltpu.async_copy(r_hbm.at[cid].at[dslab.at[c]], RD[b], GS[b])
            pltpu.async_copy(h_hbm.at[cid].at[SB[b]], HS[b], GS[b])

        def wait_g(b):
            pltpu.make_async_copy(q_hbm.at[cid].at[SB[b]], QS[b], GS[b]).wait()
            pltpu.make_async_copy(r_hbm.at[cid].at[SB[b]], RD[b], GS[b]).wait()
            pltpu.make_async_copy(h_hbm.at[cid].at[SB[b]], HS[b], GS[b]).wait()

        def wait_s(b):
            @pl.when(cid == 0)
            def _():
                pltpu.make_async_copy(EB[b], den_sp.at[dslab.at[0]],
                                      SS[b]).wait()

            pltpu.make_async_copy(HS[b], acc_sp.at[dslab.at[0]], SS[b]).wait()

        def compute_scatter(c, b):
            qs_b, rd_b, hs_b, eb_b = QS[b], RD[b], HS[b], EB[b]

            @pl.loop(0, CH)
            def _(e):
                a = qs_b[e, :] + rd_b[e, :]
                al = jnp.maximum(a, 0.2 * a)
                exv = jnp.exp(al - cv)
                eb_b[e, :] = exv
                for g in range(ngrp):
                    sp = jnp.full((16,), exv[g], jnp.float32)
                    hs_b[e, pl.ds(g * 16, 16)] = hs_b[e, pl.ds(g * 16, 16)] * sp

            @pl.when(cid == 0)
            def _():
                pltpu.async_copy(eb_b, den_sp.at[dslab.at[c]], SS[b], add=True)

            pltpu.async_copy(hs_b, acc_sp.at[dslab.at[c]], SS[b], add=True)

        def substep(i, b, do_ws, do_fi, do_fg):
            if do_ws:
                wait_s((b + 1) % 3)
            if do_fi:
                fire_sidx(i + 2, (b + 2) % 3)
            if do_fg:
                wait_sidx((b + 1) % 3)
                fire_g(i + 1, (b + 1) % 3)
            wait_g(b)
            compute_scatter(i, b)

        fire_sidx(0, 0)
        fire_sidx(1, 1)
        wait_sidx(0)
        fire_g(0, 0)
        substep(0, 0, False, True, True)
        substep(1, 1, False, True, True)
        substep(2, 2, True, True, True)
        substep(3, 0, True, True, True)

        @pl.loop(0, 51)
        def _(m):
            i0 = 4 + m * 3
            substep(i0, 1, True, True, True)
            substep(i0 + 1, 2, True, True, True)
            substep(i0 + 2, 0, True, True, True)

        substep(NCPT - 2, 1, True, False, True)
        substep(NCPT - 1, 2, False, False, False)
        wait_s(0)
        wait_s(1)
        wait_s(2)

        plsc.subcore_barrier()

        for j in range(5):
            i = j * 16 + sid

            @pl.when(i < NFULL)
            def _():
                pltpu.sync_copy(acc_sp.at[pl.ds(i * CH, CH)],
                                acc_out.at[cid, pl.ds(i * CH, CH)])
                pltpu.sync_copy(den_sp.at[pl.ds(i * CH, CH)],
                                den_out.at[cid, pl.ds(i * CH, CH)])

        @pl.when(sid == 15)
        def _():
            pltpu.sync_copy(acc_sp.at[pl.ds(NFULL * CH, NTAIL)],
                            acc_out.at[cid, pl.ds(NFULL * CH, NTAIL)])
            pltpu.sync_copy(den_sp.at[pl.ds(NFULL * CH, NTAIL)],
                            den_out.at[cid, pl.ds(NFULL * CH, NTAIL)])

    return k(h2s, Q2s, R2s, C2s, src2d, dst2d)


def _lrelu(x):
    return jnp.maximum(x, 0.2 * x)


def _stack_tables(Q, R, C):
    Q2s = jnp.stack([Q, jnp.roll(Q, -4, axis=1)])
    R2s = jnp.stack([R, jnp.roll(R, -4, axis=1)])
    C2s = jnp.concatenate([C, jnp.roll(C, -4, axis=1)])
    return Q2s, R2s, C2s


def kernel(x, edge_index, W1, att_src1, att_dst1, b1, W2, att_src2, att_dst2, b2):
    src = edge_index[0].astype(jnp.int32)
    dst = edge_index[1].astype(jnp.int32)
    pad = E_PAD - E
    src2d = jnp.concatenate([src, jnp.zeros((pad,), jnp.int32)]).reshape(-1, CH)
    dst2d = jnp.concatenate([dst, jnp.full((pad,), N, jnp.int32)]).reshape(-1, CH)

    lane = jnp.arange(16, dtype=jnp.int32) % 8
    grp = jnp.arange(IN_CH, dtype=jnp.int32) // HID
    onehot1 = (grp[:, None] == lane[None, :]).astype(jnp.float32)
    As1 = onehot1 * att_src1.reshape(IN_CH)[:, None]
    Ad1 = onehot1 * att_dst1.reshape(IN_CH)[:, None]
    As2 = jnp.broadcast_to(att_src2.reshape(OUT_CH)[:, None], (OUT_CH, 16))
    Ad2 = jnp.broadcast_to(att_dst2.reshape(OUT_CH)[:, None], (OUT_CH, 16))
    Exp8 = (jnp.arange(8, dtype=jnp.int32)[:, None]
            == grp[None, :]).astype(jnp.float32)

    h1, Q1, R1, QM1, RM1 = _tc1(x, W1, As1, Ad1)
    C1 = _lrelu(QM1 + RM1)
    Q1s, R1s, C1s = _stack_tables(Q1, R1, C1)
    acc1, den1 = _sc_edge_pass(h1, Q1s, R1s, C1s, src2d, dst2d, IN_CH)

    h2, S2, D2, SM2, DM2 = _tc2(acc1, den1, b1.reshape(1, IN_CH), Exp8,
                                W2, As2, Ad2)
    C2 = _lrelu(SM2 + DM2)
    S2s = jnp.stack([S2, S2])
    D2s = jnp.stack([D2, D2])
    C2s = jnp.concatenate([C2, C2])
    acc2, den2 = _sc_edge_pass(h2, S2s, D2s, C2s, src2d, dst2d, OUT_CH)

    return _tc3(acc2, den2, b2.reshape(1, OUT_CH))


# speedup vs baseline: 1.6151x; 1.6151x over previous
import functools

import jax
import jax.numpy as jnp
from jax import lax
from jax.experimental import pallas as pl
from jax.experimental.pallas import tpu as pltpu
from jax.experimental.pallas import tpu_sc as plsc

N = 10000
E = 320000
IN_CH = 128
HID = 16
HEADS = 8
OUT_CH = 64

_HIGH = jax.lax.Precision.HIGHEST
_BM = 1000
_G = N // _BM
CH = 128
NFULL = N // CH
NTAIL = N - NFULL * CH
NCPT = 159
E_PAD = 16 * NCPT * CH
NROWS = N + 8


def _sc_mesh():
    return plsc.VectorSubcoreMesh(core_axis_name="c", subcore_axis_name="s")


def _dot(a, b):
    return jnp.dot(a, b, precision=_HIGH, preferred_element_type=jnp.float32)


def _tc1_body(x_ref, w_ref, as_ref, ad_ref, h_ref, q_ref, r_ref, qm_ref, rm_ref):
    i = pl.program_id(0)
    h = _dot(x_ref[...], w_ref[...])
    h_ref[0] = h[:, :IN_CH // 2]
    h_ref[1] = h[:, IN_CH // 2:]
    q = _dot(h, as_ref[...])
    r = _dot(h, ad_ref[...])
    q_ref[...] = q
    r_ref[...] = r
    qm = jnp.max(q, axis=0, keepdims=True)
    rm = jnp.max(r, axis=0, keepdims=True)

    @pl.when(i == 0)
    def _():
        qm_ref[...] = qm
        rm_ref[...] = rm

    @pl.when(i > 0)
    def _():
        qm_ref[...] = jnp.maximum(qm_ref[...], qm)
        rm_ref[...] = jnp.maximum(rm_ref[...], rm)


def _tc2_body(acc_ref, den_ref, b1_ref, exp_ref, w2_ref, as_ref, ad_ref,
              h2_ref, s_ref, d_ref, sm_ref, dm_ref):
    i = pl.program_id(0)
    acc = jnp.concatenate([acc_ref[0], acc_ref[1]], axis=1)
    den8 = den_ref[0][:, :8] + den_ref[1][:, :8]
    den128 = _dot(den8, exp_ref[...])
    h1 = acc / (den128 + 1e-16) + b1_ref[...]
    h2 = _dot(h1, w2_ref[...])
    h2_ref[0] = h2[:, :OUT_CH // 2]
    h2_ref[1] = h2[:, OUT_CH // 2:]
    s = _dot(h2, as_ref[...])
    d = _dot(h2, ad_ref[...])
    s_ref[...] = s
    d_ref[...] = d
    sm = jnp.max(s, axis=0, keepdims=True)
    dm = jnp.max(d, axis=0, keepdims=True)

    @pl.when(i == 0)
    def _():
        sm_ref[...] = sm
        dm_ref[...] = dm

    @pl.when(i > 0)
    def _():
        sm_ref[...] = jnp.maximum(sm_ref[...], sm)
        dm_ref[...] = jnp.maximum(dm_ref[...], dm)


def _tc3_body(acc_ref, den_ref, b2_ref, out_ref):
    acc = jnp.concatenate([acc_ref[0], acc_ref[1]], axis=1)
    den = den_ref[0][:, 0:1] + den_ref[1][:, 0:1]
    o = acc / (den + 1e-16) + b2_ref[...]
    m = jnp.max(o, axis=1, keepdims=True)
    z = o - m
    lse = jnp.log(jnp.sum(jnp.exp(z), axis=1, keepdims=True))
    out_ref[...] = z - lse


def _tc1(x, W1, As1, Ad1):
    return pl.pallas_call(
        _tc1_body,
        grid=(_G,),
        in_specs=[
            pl.BlockSpec((_BM, IN_CH), lambda i: (i, 0)),
            pl.BlockSpec((IN_CH, IN_CH), lambda i: (0, 0)),
            pl.BlockSpec((IN_CH, 16), lambda i: (0, 0)),
            pl.BlockSpec((IN_CH, 16), lambda i: (0, 0)),
        ],
        out_specs=[
            pl.BlockSpec((2, _BM, IN_CH // 2), lambda i: (0, i, 0)),
            pl.BlockSpec((_BM, 16), lambda i: (i, 0)),
            pl.BlockSpec((_BM, 16), lambda i: (i, 0)),
            pl.BlockSpec((1, 16), lambda i: (0, 0)),
            pl.BlockSpec((1, 16), lambda i: (0, 0)),
        ],
        out_shape=[
            jax.ShapeDtypeStruct((2, N, IN_CH // 2), jnp.float32),
            jax.ShapeDtypeStruct((N, 16), jnp.float32),
            jax.ShapeDtypeStruct((N, 16), jnp.float32),
            jax.ShapeDtypeStruct((1, 16), jnp.float32),
            jax.ShapeDtypeStruct((1, 16), jnp.float32),
        ],
    )(x, W1, As1, Ad1)


def _tc2(acc1, den1, b1, Exp8, W2, As2, Ad2):
    return pl.pallas_call(
        _tc2_body,
        grid=(_G,),
        in_specs=[
            pl.BlockSpec((2, _BM, IN_CH // 2), lambda i: (0, i, 0)),
            pl.BlockSpec((2, _BM, 16), lambda i: (0, i, 0)),
            pl.BlockSpec((1, IN_CH), lambda i: (0, 0)),
            pl.BlockSpec((8, IN_CH), lambda i: (0, 0)),
            pl.BlockSpec((IN_CH, OUT_CH), lambda i: (0, 0)),
            pl.BlockSpec((OUT_CH, 16), lambda i: (0, 0)),
            pl.BlockSpec((OUT_CH, 16), lambda i: (0, 0)),
        ],
        out_specs=[
            pl.BlockSpec((2, _BM, OUT_CH // 2), lambda i: (0, i, 0)),
            pl.BlockSpec((_BM, 16), lambda i: (i, 0)),
            pl.BlockSpec((_BM, 16), lambda i: (i, 0)),
            pl.BlockSpec((1, 16), lambda i: (0, 0)),
            pl.BlockSpec((1, 16), lambda i: (0, 0)),
        ],
        out_shape=[
            jax.ShapeDtypeStruct((2, N, OUT_CH // 2), jnp.float32),
            jax.ShapeDtypeStruct((N, 16), jnp.float32),
            jax.ShapeDtypeStruct((N, 16), jnp.float32),
            jax.ShapeDtypeStruct((1, 16), jnp.float32),
            jax.ShapeDtypeStruct((1, 16), jnp.float32),
        ],
    )(acc1, den1, b1, Exp8, W2, As2, Ad2)


def _tc3(acc2, den2, b2):
    return pl.pallas_call(
        _tc3_body,
        grid=(_G,),
        in_specs=[
            pl.BlockSpec((2, _BM, OUT_CH // 2), lambda i: (0, i, 0)),
            pl.BlockSpec((2, _BM, 16), lambda i: (0, i, 0)),
            pl.BlockSpec((1, OUT_CH), lambda i: (0, 0)),
        ],
        out_specs=pl.BlockSpec((_BM, OUT_CH), lambda i: (i, 0)),
        out_shape=jax.ShapeDtypeStruct((N, OUT_CH), jnp.float32),
    )(acc2, den2, b2)


def _sc_edge_pass(h2s, Q2s, R2s, C2s, src2d, dst2d, F):
    FH = F // 2
    ngrp = FH // 16

    @functools.partial(
        pl.kernel,
        out_type=[
            jax.ShapeDtypeStruct((2, N, FH), jnp.float32),
            jax.ShapeDtypeStruct((2, N, 16), jnp.float32),
        ],
        mesh=_sc_mesh(),
        compiler_params=pltpu.CompilerParams(use_tc_tiling_on_sc=False),
        scratch_types=[
            pltpu.VMEM((NCPT, CH), jnp.int32),
            [pltpu.VMEM((CH,), jnp.int32)] * 3,
            [pltpu.VMEM((CH, 16), jnp.float32)] * 3,
            [pltpu.VMEM((CH, 16), jnp.float32)] * 3,
            [pltpu.VMEM((CH, FH), jnp.float32)] * 3,
            [pltpu.VMEM((CH, 16), jnp.float32)] * 3,
            pltpu.VMEM((1, 16), jnp.float32),
            pltpu.VMEM_SHARED((NROWS, FH), jnp.float32),
            pltpu.VMEM_SHARED((NROWS, 16), jnp.float32),
            [pltpu.SemaphoreType.DMA] * 3,
            [pltpu.SemaphoreType.DMA] * 3,
            [pltpu.SemaphoreType.DMA] * 3,
        ],
    )
    def k(h_hbm, q_hbm, r_hbm, c_hbm, s_hbm, d_hbm, acc_out, den_out,
          dslab, SB, QS, RD, HS, EB, cvec, acc_sp, den_sp, IS, GS, SS):
        cid = lax.axis_index("c")
        sid = lax.axis_index("s")

        hs0, eb0 = HS[0], EB[0]

        @pl.loop(0, CH)
        def _(r2):
            for j in range(ngrp):
                hs0[r2, pl.ds(j * 16, 16)] = jnp.zeros((16,), jnp.float32)
            eb0[r2, :] = jnp.zeros((16,), jnp.float32)

        for j in range(5):
            i = j * 16 + sid

            @pl.when(i < NFULL)
            def _():
                pltpu.sync_copy(hs0, acc_sp.at[pl.ds(i * CH, CH)])
                pltpu.sync_copy(eb0, den_sp.at[pl.ds(i * CH, CH)])

        @pl.when(sid == 15)
        def _():
            pltpu.sync_copy(hs0.at[pl.ds(0, NTAIL)],
                            acc_sp.at[pl.ds(NFULL * CH, NTAIL)])
            pltpu.sync_copy(eb0.at[pl.ds(0, NTAIL)],
                            den_sp.at[pl.ds(NFULL * CH, NTAIL)])

        pltpu.sync_copy(c_hbm.at[pl.ds(cid, 1)], cvec)
        pltpu.sync_copy(d_hbm.at[pl.ds(sid * NCPT, NCPT)], dslab)
        plsc.subcore_barrier()
        cv = cvec[0, :]
        base = sid * NCPT

        def fire_sidx(c, b):
            pltpu.async_copy(s_hbm.at[base + c], SB[b], IS[b])

        def wait_sidx(b):
            pltpu.make_async_copy(s_hbm.at[base], SB[b], IS[b]).wait()

        def fire_g(c, b):
            pltpu.async_copy(q_hbm.at[cid].at[SB[b]], QS[b], GS[b])
            pltpu.async_copy(r_hbm.at[cid].at[dslab.at[c]], RD[b], GS[b])
            pltpu.async_copy(h_hbm.at[cid].at[SB[b]], HS[b], GS[b])

        def wait_g(b):
            pltpu.make_async_copy(q_hbm.at[cid].at[SB[b]], QS[b], GS[b]).wait()
            pltpu.make_async_copy(r_hbm.at[cid].at[SB[b]], RD[b], GS[b]).wait()
            pltpu.make_async_copy(h_hbm.at[cid].at[SB[b]], HS[b], GS[b]).wait()

        def wait_s(b):
            @pl.when(cid == 0)
            def _():
                pltpu.make_async_copy(EB[b], den_sp.at[dslab.at[0]],
                                      SS[b]).wait()

            pltpu.make_async_copy(HS[b], acc_sp.at[dslab.at[0]], SS[b]).wait()

        def compute_scatter(c, b):
            qs_b, rd_b, hs_b, eb_b = QS[b], RD[b], HS[b], EB[b]

            @plsc.parallel_loop(0, CH, unroll=4)
            def _(e):
                a = qs_b[e, :] + rd_b[e, :]
                al = jnp.maximum(a, 0.2 * a)
                exv = jnp.exp(al - cv)
                eb_b[e, :] = exv
                for g in range(ngrp):
                    sp = jnp.full((16,), exv[g], jnp.float32)
                    hs_b[e, pl.ds(g * 16, 16)] = hs_b[e, pl.ds(g * 16, 16)] * sp

            @pl.when(cid == 0)
            def _():
                pltpu.async_copy(eb_b, den_sp.at[dslab.at[c]], SS[b], add=True)

            pltpu.async_copy(hs_b, acc_sp.at[dslab.at[c]], SS[b], add=True)

        def substep(i, b, do_ws, do_fi, do_fg):
            if do_ws:
                wait_s((b + 1) % 3)
            if do_fi:
                fire_sidx(i + 2, (b + 2) % 3)
            if do_fg:
                wait_sidx((b + 1) % 3)
                fire_g(i + 1, (b + 1) % 3)
            wait_g(b)
            compute_scatter(i, b)

        fire_sidx(0, 0)
        fire_sidx(1, 1)
        wait_sidx(0)
        fire_g(0, 0)
        substep(0, 0, False, True, True)
        substep(1, 1, False, True, True)
        substep(2, 2, True, True, True)
        substep(3, 0, True, True, True)

        @pl.loop(0, 51)
        def _(m):
            i0 = 4 + m * 3
            substep(i0, 1, True, True, True)
            substep(i0 + 1, 2, True, True, True)
            substep(i0 + 2, 0, True, True, True)

        substep(NCPT - 2, 1, True, False, True)
        substep(NCPT - 1, 2, False, False, False)
        wait_s(0)
        wait_s(1)
        wait_s(2)

        plsc.subcore_barrier()

        for j in range(5):
            i = j * 16 + sid

            @pl.when(i < NFULL)
            def _():
                pltpu.sync_copy(acc_sp.at[pl.ds(i * CH, CH)],
                                acc_out.at[cid, pl.ds(i * CH, CH)])
                pltpu.sync_copy(den_sp.at[pl.ds(i * CH, CH)],
                                den_out.at[cid, pl.ds(i * CH, CH)])

        @pl.when(sid == 15)
        def _():
            pltpu.sync_copy(acc_sp.at[pl.ds(NFULL * CH, NTAIL)],
                            acc_out.at[cid, pl.ds(NFULL * CH, NTAIL)])
            pltpu.sync_copy(den_sp.at[pl.ds(NFULL * CH, NTAIL)],
                            den_out.at[cid, pl.ds(NFULL * CH, NTAIL)])

    return k(h2s, Q2s, R2s, C2s, src2d, dst2d)


def _lrelu(x):
    return jnp.maximum(x, 0.2 * x)


def _stack_tables(Q, R, C):
    Q2s = jnp.stack([Q, jnp.roll(Q, -4, axis=1)])
    R2s = jnp.stack([R, jnp.roll(R, -4, axis=1)])
    C2s = jnp.concatenate([C, jnp.roll(C, -4, axis=1)])
    return Q2s, R2s, C2s


def kernel(x, edge_index, W1, att_src1, att_dst1, b1, W2, att_src2, att_dst2, b2):
    src = edge_index[0].astype(jnp.int32)
    dst = edge_index[1].astype(jnp.int32)
    pad = E_PAD - E
    src2d = jnp.concatenate([src, jnp.zeros((pad,), jnp.int32)]).reshape(-1, CH)
    dst2d = jnp.concatenate([dst, jnp.full((pad,), N, jnp.int32)]).reshape(-1, CH)

    lane = jnp.arange(16, dtype=jnp.int32) % 8
    grp = jnp.arange(IN_CH, dtype=jnp.int32) // HID
    onehot1 = (grp[:, None] == lane[None, :]).astype(jnp.float32)
    As1 = onehot1 * att_src1.reshape(IN_CH)[:, None]
    Ad1 = onehot1 * att_dst1.reshape(IN_CH)[:, None]
    As2 = jnp.broadcast_to(att_src2.reshape(OUT_CH)[:, None], (OUT_CH, 16))
    Ad2 = jnp.broadcast_to(att_dst2.reshape(OUT_CH)[:, None], (OUT_CH, 16))
    Exp8 = (jnp.arange(8, dtype=jnp.int32)[:, None]
            == grp[None, :]).astype(jnp.float32)

    h1, Q1, R1, QM1, RM1 = _tc1(x, W1, As1, Ad1)
    C1 = _lrelu(QM1 + RM1)
    Q1s, R1s, C1s = _stack_tables(Q1, R1, C1)
    acc1, den1 = _sc_edge_pass(h1, Q1s, R1s, C1s, src2d, dst2d, IN_CH)

    h2, S2, D2, SM2, DM2 = _tc2(acc1, den1, b1.reshape(1, IN_CH), Exp8,
                                W2, As2, Ad2)
    C2 = _lrelu(SM2 + DM2)
    S2s = jnp.stack([S2, S2])
    D2s = jnp.stack([D2, D2])
    C2s = jnp.concatenate([C2, C2])
    acc2, den2 = _sc_edge_pass(h2, S2s, D2s, C2s, src2d, dst2d, OUT_CH)

    return _tc3(acc2, den2, b2.reshape(1, OUT_CH))


# speedup vs baseline: 1.6306x; 1.0096x over previous
import functools

import jax
import jax.numpy as jnp
from jax import lax
from jax.experimental import pallas as pl
from jax.experimental.pallas import tpu as pltpu
from jax.experimental.pallas import tpu_sc as plsc

N = 10000
E = 320000
IN_CH = 128
HID = 16
HEADS = 8
OUT_CH = 64

_HIGH = jax.lax.Precision.HIGHEST
_BM = 1000
_G = N // _BM
CH = 128
NFULL = N // CH
NTAIL = N - NFULL * CH
NCPT = 159
E_PAD = 16 * NCPT * CH
NROWS = N + 8


def _sc_mesh():
    return plsc.VectorSubcoreMesh(core_axis_name="c", subcore_axis_name="s")


def _dot(a, b):
    return jnp.dot(a, b, precision=_HIGH, preferred_element_type=jnp.float32)


def _roll4(v):
    return jnp.concatenate([v[:, 4:], v[:, :4]], axis=1)


def _onehot_heads():
    lane = jax.lax.broadcasted_iota(jnp.int32, (IN_CH, 16), 1) % 8
    grp = jax.lax.broadcasted_iota(jnp.int32, (IN_CH, 16), 0) // HID
    return jnp.where(grp == lane, 1.0, 0.0).astype(jnp.float32)


def _tc1_body(x_ref, w_ref, as_ref, ad_ref, h_ref, q_ref, r_ref, qm_ref,
              rm_ref, c_ref):
    i = pl.program_id(0)
    h = _dot(x_ref[...], w_ref[...])
    h_ref[0] = h[:, :IN_CH // 2]
    h_ref[1] = h[:, IN_CH // 2:]
    oneh = _onehot_heads()
    q = _dot(h * as_ref[...], oneh)
    r = _dot(h * ad_ref[...], oneh)
    q_ref[0] = q
    q_ref[1] = _roll4(q)
    r_ref[0] = r
    r_ref[1] = _roll4(r)
    qm = jnp.max(q, axis=0, keepdims=True)
    rm = jnp.max(r, axis=0, keepdims=True)

    @pl.when(i == 0)
    def _():
        qm_ref[...] = qm
        rm_ref[...] = rm

    @pl.when(i > 0)
    def _():
        qm_ref[...] = jnp.maximum(qm_ref[...], qm)
        rm_ref[...] = jnp.maximum(rm_ref[...], rm)

    @pl.when(i == _G - 1)
    def _():
        cs = qm_ref[...] + rm_ref[...]
        c = jnp.maximum(cs, 0.2 * cs)
        c_ref[...] = jnp.concatenate([c, _roll4(c)], axis=0)


def _tc2_body(acc_ref, den_ref, b1_ref, w2_ref, as_ref, ad_ref,
              h2_ref, s_ref, d_ref, sm_ref, dm_ref, c_ref):
    i = pl.program_id(0)
    acc = jnp.concatenate([acc_ref[0], acc_ref[1]], axis=1)
    den8 = den_ref[0][:, :8] + den_ref[1][:, :8]
    exp8 = jnp.where(
        jax.lax.broadcasted_iota(jnp.int32, (8, IN_CH), 0)
        == jax.lax.broadcasted_iota(jnp.int32, (8, IN_CH), 1) // HID,
        1.0, 0.0).astype(jnp.float32)
    den128 = _dot(den8, exp8)
    h1 = acc / (den128 + 1e-16) + b1_ref[...]
    h2 = _dot(h1, w2_ref[...])
    h2_ref[0] = h2[:, :OUT_CH // 2]
    h2_ref[1] = h2[:, OUT_CH // 2:]
    s = jnp.broadcast_to(_dot(h2, as_ref[...]), (h2.shape[0], 16))
    d = jnp.broadcast_to(_dot(h2, ad_ref[...]), (h2.shape[0], 16))
    s_ref[0] = s
    s_ref[1] = s
    d_ref[0] = d
    d_ref[1] = d
    sm = jnp.max(s, axis=0, keepdims=True)
    dm = jnp.max(d, axis=0, keepdims=True)

    @pl.when(i == 0)
    def _():
        sm_ref[...] = sm
        dm_ref[...] = dm

    @pl.when(i > 0)
    def _():
        sm_ref[...] = jnp.maximum(sm_ref[...], sm)
        dm_ref[...] = jnp.maximum(dm_ref[...], dm)

    @pl.when(i == _G - 1)
    def _():
        cs = sm_ref[...] + dm_ref[...]
        c = jnp.maximum(cs, 0.2 * cs)
        c_ref[...] = jnp.concatenate([c, c], axis=0)


def _tc3_body(acc_ref, den_ref, b2_ref, out_ref):
    acc = jnp.concatenate([acc_ref[0], acc_ref[1]], axis=1)
    den = den_ref[0][:, 0:1] + den_ref[1][:, 0:1]
    o = acc / (den + 1e-16) + b2_ref[...]
    m = jnp.max(o, axis=1, keepdims=True)
    z = o - m
    lse = jnp.log(jnp.sum(jnp.exp(z), axis=1, keepdims=True))
    out_ref[...] = z - lse


def _tc1(x, W1, av1, adv1):
    return pl.pallas_call(
        _tc1_body,
        grid=(_G,),
        in_specs=[
            pl.BlockSpec((_BM, IN_CH), lambda i: (i, 0)),
            pl.BlockSpec((IN_CH, IN_CH), lambda i: (0, 0)),
            pl.BlockSpec((1, IN_CH), lambda i: (0, 0)),
            pl.BlockSpec((1, IN_CH), lambda i: (0, 0)),
        ],
        out_specs=[
            pl.BlockSpec((2, _BM, IN_CH // 2), lambda i: (0, i, 0)),
            pl.BlockSpec((2, _BM, 16), lambda i: (0, i, 0)),
            pl.BlockSpec((2, _BM, 16), lambda i: (0, i, 0)),
            pl.BlockSpec((1, 16), lambda i: (0, 0)),
            pl.BlockSpec((1, 16), lambda i: (0, 0)),
            pl.BlockSpec((2, 16), lambda i: (0, 0)),
        ],
        out_shape=[
            jax.ShapeDtypeStruct((2, N, IN_CH // 2), jnp.float32),
            jax.ShapeDtypeStruct((2, N, 16), jnp.float32),
            jax.ShapeDtypeStruct((2, N, 16), jnp.float32),
            jax.ShapeDtypeStruct((1, 16), jnp.float32),
            jax.ShapeDtypeStruct((1, 16), jnp.float32),
            jax.ShapeDtypeStruct((2, 16), jnp.float32),
        ],
    )(x, W1, av1, adv1)


def _tc2(acc1, den1, b1, W2, as2, ad2):
    return pl.pallas_call(
        _tc2_body,
        grid=(_G,),
        in_specs=[
            pl.BlockSpec((2, _BM, IN_CH // 2), lambda i: (0, i, 0)),
            pl.BlockSpec((2, _BM, 16), lambda i: (0, i, 0)),
            pl.BlockSpec((1, IN_CH), lambda i: (0, 0)),
            pl.BlockSpec((IN_CH, OUT_CH), lambda i: (0, 0)),
            pl.BlockSpec((OUT_CH, 1), lambda i: (0, 0)),
            pl.BlockSpec((OUT_CH, 1), lambda i: (0, 0)),
        ],
        out_specs=[
            pl.BlockSpec((2, _BM, OUT_CH // 2), lambda i: (0, i, 0)),
            pl.BlockSpec((2, _BM, 16), lambda i: (0, i, 0)),
            pl.BlockSpec((2, _BM, 16), lambda i: (0, i, 0)),
            pl.BlockSpec((1, 16), lambda i: (0, 0)),
            pl.BlockSpec((1, 16), lambda i: (0, 0)),
            pl.BlockSpec((2, 16), lambda i: (0, 0)),
        ],
        out_shape=[
            jax.ShapeDtypeStruct((2, N, OUT_CH // 2), jnp.float32),
            jax.ShapeDtypeStruct((2, N, 16), jnp.float32),
            jax.ShapeDtypeStruct((2, N, 16), jnp.float32),
            jax.ShapeDtypeStruct((1, 16), jnp.float32),
            jax.ShapeDtypeStruct((1, 16), jnp.float32),
            jax.ShapeDtypeStruct((2, 16), jnp.float32),
        ],
    )(acc1, den1, b1, W2, as2, ad2)


def _tc3(acc2, den2, b2):
    return pl.pallas_call(
        _tc3_body,
        grid=(_G,),
        in_specs=[
            pl.BlockSpec((2, _BM, OUT_CH // 2), lambda i: (0, i, 0)),
            pl.BlockSpec((2, _BM, 16), lambda i: (0, i, 0)),
            pl.BlockSpec((1, OUT_CH), lambda i: (0, 0)),
        ],
        out_specs=pl.BlockSpec((_BM, OUT_CH), lambda i: (i, 0)),
        out_shape=jax.ShapeDtypeStruct((N, OUT_CH), jnp.float32),
    )(acc2, den2, b2)


def _sc_edge_pass(h2s, Q2s, R2s, C2s, src2d, dst2d, F):
    FH = F // 2
    ngrp = FH // 16

    @functools.partial(
        pl.kernel,
        out_type=[
            jax.ShapeDtypeStruct((2, N, FH), jnp.float32),
            jax.ShapeDtypeStruct((2, N, 16), jnp.float32),
        ],
        mesh=_sc_mesh(),
        compiler_params=pltpu.CompilerParams(use_tc_tiling_on_sc=False),
        scratch_types=[
            pltpu.VMEM((NCPT, CH), jnp.int32),
            [pltpu.VMEM((CH,), jnp.int32)] * 3,
            [pltpu.VMEM((CH, 16), jnp.float32)] * 3,
            [pltpu.VMEM((CH, 16), jnp.float32)] * 3,
            [pltpu.VMEM((CH, FH), jnp.float32)] * 3,
            [pltpu.VMEM((CH, 16), jnp.float32)] * 3,
            pltpu.VMEM((1, 16), jnp.float32),
            pltpu.VMEM_SHARED((NROWS, FH), jnp.float32),
            pltpu.VMEM_SHARED((NROWS, 16), jnp.float32),
            [pltpu.SemaphoreType.DMA] * 3,
            [pltpu.SemaphoreType.DMA] * 3,
            [pltpu.SemaphoreType.DMA] * 3,
            [pltpu.SemaphoreType.DMA] * 3,
        ],
    )
    def k(h_hbm, q_hbm, r_hbm, c_hbm, s_hbm, d_hbm, acc_out, den_out,
          dslab, SB, QS, RD, HS, EB, cvec, acc_sp, den_sp, IS, GS, SS, DS):
        cid = lax.axis_index("c")
        sid = lax.axis_index("s")

        hs0, eb0 = HS[0], EB[0]

        @pl.loop(0, CH)
        def _(r2):
            for j in range(ngrp):
                hs0[r2, pl.ds(j * 16, 16)] = jnp.zeros((16,), jnp.float32)
            eb0[r2, :] = jnp.zeros((16,), jnp.float32)

        for j in range(5):
            i = j * 16 + sid

            @pl.when(i < NFULL)
            def _():
                pltpu.sync_copy(hs0, acc_sp.at[pl.ds(i * CH, CH)])
                pltpu.sync_copy(eb0, den_sp.at[pl.ds(i * CH, CH)])

        @pl.when(sid == 15)
        def _():
            pltpu.sync_copy(hs0.at[pl.ds(0, NTAIL)],
                            acc_sp.at[pl.ds(NFULL * CH, NTAIL)])
            pltpu.sync_copy(eb0.at[pl.ds(0, NTAIL)],
                            den_sp.at[pl.ds(NFULL * CH, NTAIL)])

        pltpu.sync_copy(c_hbm.at[pl.ds(cid, 1)], cvec)
        pltpu.sync_copy(d_hbm.at[pl.ds(sid * NCPT, NCPT)], dslab)
        plsc.subcore_barrier()
        cv = cvec[0, :]
        base = sid * NCPT

        def fire_sidx(c, b):
            pltpu.async_copy(s_hbm.at[base + c], SB[b], IS[b])

        def wait_sidx(b):
            pltpu.make_async_copy(s_hbm.at[base], SB[b], IS[b]).wait()

        def fire_g(c, b):
            pltpu.async_copy(q_hbm.at[cid].at[SB[b]], QS[b], GS[b])
            pltpu.async_copy(r_hbm.at[cid].at[dslab.at[c]], RD[b], GS[b])
            pltpu.async_copy(h_hbm.at[cid].at[SB[b]], HS[b], GS[b])

        def wait_g(b):
            pltpu.make_async_copy(q_hbm.at[cid].at[SB[b]], QS[b], GS[b]).wait()
            pltpu.make_async_copy(r_hbm.at[cid].at[SB[b]], RD[b], GS[b]).wait()
            pltpu.make_async_copy(h_hbm.at[cid].at[SB[b]], HS[b], GS[b]).wait()

        def wait_s(b, c):
            @pl.when((c % 2) == cid)
            def _():
                pltpu.make_async_copy(EB[b], den_sp.at[dslab.at[0]],
                                      DS[b]).wait()

            pltpu.make_async_copy(HS[b], acc_sp.at[dslab.at[0]], SS[b]).wait()

        def compute_scatter(c, b):
            qs_b, rd_b, hs_b, eb_b = QS[b], RD[b], HS[b], EB[b]

            @plsc.parallel_loop(0, CH, unroll=4)
            def _(e):
                a = qs_b[e, :] + rd_b[e, :]
                al = jnp.maximum(a, 0.2 * a)
                exv = jnp.exp(al - cv)
                eb_b[e, :] = exv
                for g in range(ngrp):
                    sp = jnp.full((16,), exv[g], jnp.float32)
                    hs_b[e, pl.ds(g * 16, 16)] = hs_b[e, pl.ds(g * 16, 16)] * sp

            @pl.when((c % 2) == cid)
            def _():
                pltpu.async_copy(eb_b, den_sp.at[dslab.at[c]], DS[b], add=True)

            pltpu.async_copy(hs_b, acc_sp.at[dslab.at[c]], SS[b], add=True)

        def substep(i, b, do_ws, do_fi, do_fg):
            if do_ws:
                wait_s((b + 1) % 3, i - 2)
            if do_fi:
                fire_sidx(i + 2, (b + 2) % 3)
            if do_fg:
                wait_sidx((b + 1) % 3)
                fire_g(i + 1, (b + 1) % 3)
            wait_g(b)
            compute_scatter(i, b)

        fire_sidx(0, 0)
        fire_sidx(1, 1)
        wait_sidx(0)
        fire_g(0, 0)
        substep(0, 0, False, True, True)
        substep(1, 1, False, True, True)
        substep(2, 2, True, True, True)
        substep(3, 0, True, True, True)

        @pl.loop(0, 51)
        def _(m):
            i0 = 4 + m * 3
            substep(i0, 1, True, True, True)
            substep(i0 + 1, 2, True, True, True)
            substep(i0 + 2, 0, True, True, True)

        substep(NCPT - 2, 1, True, False, True)
        substep(NCPT - 1, 2, False, False, False)
        wait_s(0, NCPT - 3)
        wait_s(1, NCPT - 2)
        wait_s(2, NCPT - 1)

        plsc.subcore_barrier()

        for j in range(5):
            i = j * 16 + sid

            @pl.when(i < NFULL)
            def _():
                pltpu.sync_copy(acc_sp.at[pl.ds(i * CH, CH)],
                                acc_out.at[cid, pl.ds(i * CH, CH)])
                pltpu.sync_copy(den_sp.at[pl.ds(i * CH, CH)],
                                den_out.at[cid, pl.ds(i * CH, CH)])

        @pl.when(sid == 15)
        def _():
            pltpu.sync_copy(acc_sp.at[pl.ds(NFULL * CH, NTAIL)],
                            acc_out.at[cid, pl.ds(NFULL * CH, NTAIL)])
            pltpu.sync_copy(den_sp.at[pl.ds(NFULL * CH, NTAIL)],
                            den_out.at[cid, pl.ds(NFULL * CH, NTAIL)])

    return k(h2s, Q2s, R2s, C2s, src2d, dst2d)


def kernel(x, edge_index, W1, att_src1, att_dst1, b1, W2, att_src2, att_dst2, b2):
    src = edge_index[0].astype(jnp.int32)
    dst = edge_index[1].astype(jnp.int32)
    pad = E_PAD - E
    src2d = jnp.concatenate([src, jnp.zeros((pad,), jnp.int32)]).reshape(-1, CH)
    dst2d = jnp.concatenate([dst, jnp.full((pad,), N, jnp.int32)]).reshape(-1, CH)

    h1, Q1s, R1s, _, _, C1s = _tc1(x, W1, att_src1.reshape(1, IN_CH),
                                   att_dst1.reshape(1, IN_CH))
    acc1, den1 = _sc_edge_pass(h1, Q1s, R1s, C1s, src2d, dst2d, IN_CH)

    h2, S2s, D2s, _, _, C2s = _tc2(acc1, den1, b1.reshape(1, IN_CH), W2,
                                   att_src2.reshape(OUT_CH, 1),
                                   att_dst2.reshape(OUT_CH, 1))
    acc2, den2 = _sc_edge_pass(h2, S2s, D2s, C2s, src2d, dst2d, OUT_CH)

    return _tc3(acc2, den2, b2.reshape(1, OUT_CH))


# speedup vs baseline: 1.8506x; 1.1349x over previous
import functools

import jax
import jax.numpy as jnp
from jax import lax
from jax.experimental import pallas as pl
from jax.experimental.pallas import tpu as pltpu
from jax.experimental.pallas import tpu_sc as plsc

N = 10000
E = 320000
IN_CH = 128
HID = 16
HEADS = 8
OUT_CH = 64

_HIGH = jax.lax.Precision.HIGHEST
_BM = 1000
_G = N // _BM
CH = 128
NFULL = N // CH
NTAIL = N - NFULL * CH
NCPT = 159
E_PAD = 16 * NCPT * CH
NROWS = N + 8


def _sc_mesh():
    return plsc.VectorSubcoreMesh(core_axis_name="c", subcore_axis_name="s")


def _dot(a, b):
    return jnp.dot(a, b, precision=_HIGH, preferred_element_type=jnp.float32)


def _roll4(v):
    return jnp.concatenate([v[:, 4:], v[:, :4]], axis=1)


def _pack_perm(w):
    J = jax.lax.broadcasted_iota(jnp.int32, (w, w), 1)
    T = jax.lax.broadcasted_iota(jnp.int32, (w, w), 0)
    tj = 32 * (J // 32) + (J % 32) // 2 + 16 * (J % 2)
    return jnp.where(T == tj, 1.0, 0.0).astype(jnp.float32)


def _onehot_heads():
    lane = jax.lax.broadcasted_iota(jnp.int32, (IN_CH, 16), 1) % 8
    grp = jax.lax.broadcasted_iota(jnp.int32, (IN_CH, 16), 0) // HID
    return jnp.where(grp == lane, 1.0, 0.0).astype(jnp.float32)


def _tc1_body(x_ref, w_ref, as_ref, ad_ref, h_ref, q_ref, r_ref, qm_ref,
              rm_ref, c_ref):
    i = pl.program_id(0)
    h = _dot(x_ref[...], w_ref[...])
    perm = _pack_perm(IN_CH // 2)
    h_ref[0] = _dot(h[:, :IN_CH // 2], perm).astype(jnp.bfloat16)
    h_ref[1] = _dot(h[:, IN_CH // 2:], perm).astype(jnp.bfloat16)
    oneh = _onehot_heads()
    q = _dot(h * as_ref[...], oneh)
    r = _dot(h * ad_ref[...], oneh)
    q_ref[0] = q
    q_ref[1] = _roll4(q)
    r_ref[0] = r
    r_ref[1] = _roll4(r)
    qm = jnp.max(q, axis=0, keepdims=True)
    rm = jnp.max(r, axis=0, keepdims=True)

    @pl.when(i == 0)
    def _():
        qm_ref[...] = qm
        rm_ref[...] = rm

    @pl.when(i > 0)
    def _():
        qm_ref[...] = jnp.maximum(qm_ref[...], qm)
        rm_ref[...] = jnp.maximum(rm_ref[...], rm)

    @pl.when(i == _G - 1)
    def _():
        cs = qm_ref[...] + rm_ref[...]
        c = jnp.maximum(cs, 0.2 * cs)
        c_ref[...] = jnp.concatenate([c, _roll4(c)], axis=0)


def _tc2_body(acc_ref, den_ref, b1_ref, w2_ref, as_ref, ad_ref,
              h2_ref, s_ref, d_ref, sm_ref, dm_ref, c_ref):
    i = pl.program_id(0)
    acc = jnp.concatenate([acc_ref[0], acc_ref[1]], axis=1)
    den8 = den_ref[0][:, :8] + den_ref[1][:, :8]
    exp8 = jnp.where(
        jax.lax.broadcasted_iota(jnp.int32, (8, IN_CH), 0)
        == jax.lax.broadcasted_iota(jnp.int32, (8, IN_CH), 1) // HID,
        1.0, 0.0).astype(jnp.float32)
    den128 = _dot(den8, exp8)
    h1 = acc / (den128 + 1e-16) + b1_ref[...]
    h2 = _dot(h1, w2_ref[...])
    perm = _pack_perm(OUT_CH // 2)
    h2_ref[0] = _dot(h2[:, :OUT_CH // 2], perm).astype(jnp.bfloat16)
    h2_ref[1] = _dot(h2[:, OUT_CH // 2:], perm).astype(jnp.bfloat16)
    s = jnp.broadcast_to(_dot(h2, as_ref[...]), (h2.shape[0], 16))
    d = jnp.broadcast_to(_dot(h2, ad_ref[...]), (h2.shape[0], 16))
    s_ref[0] = s
    s_ref[1] = s
    d_ref[0] = d
    d_ref[1] = d
    sm = jnp.max(s, axis=0, keepdims=True)
    dm = jnp.max(d, axis=0, keepdims=True)

    @pl.when(i == 0)
    def _():
        sm_ref[...] = sm
        dm_ref[...] = dm

    @pl.when(i > 0)
    def _():
        sm_ref[...] = jnp.maximum(sm_ref[...], sm)
        dm_ref[...] = jnp.maximum(dm_ref[...], dm)

    @pl.when(i == _G - 1)
    def _():
        cs = sm_ref[...] + dm_ref[...]
        c = jnp.maximum(cs, 0.2 * cs)
        c_ref[...] = jnp.concatenate([c, c], axis=0)


def _tc3_body(acc_ref, den_ref, b2_ref, out_ref):
    acc = jnp.concatenate([acc_ref[0], acc_ref[1]], axis=1)
    den = den_ref[0][:, 0:1] + den_ref[1][:, 0:1]
    o = acc / (den + 1e-16) + b2_ref[...]
    m = jnp.max(o, axis=1, keepdims=True)
    z = o - m
    lse = jnp.log(jnp.sum(jnp.exp(z), axis=1, keepdims=True))
    out_ref[...] = z - lse


def _tc1(x, W1, av1, adv1):
    return pl.pallas_call(
        _tc1_body,
        grid=(_G,),
        in_specs=[
            pl.BlockSpec((_BM, IN_CH), lambda i: (i, 0)),
            pl.BlockSpec((IN_CH, IN_CH), lambda i: (0, 0)),
            pl.BlockSpec((1, IN_CH), lambda i: (0, 0)),
            pl.BlockSpec((1, IN_CH), lambda i: (0, 0)),
        ],
        out_specs=[
            pl.BlockSpec((2, _BM, IN_CH // 2), lambda i: (0, i, 0)),
            pl.BlockSpec((2, _BM, 16), lambda i: (0, i, 0)),
            pl.BlockSpec((2, _BM, 16), lambda i: (0, i, 0)),
            pl.BlockSpec((1, 16), lambda i: (0, 0)),
            pl.BlockSpec((1, 16), lambda i: (0, 0)),
            pl.BlockSpec((2, 16), lambda i: (0, 0)),
        ],
        out_shape=[
            jax.ShapeDtypeStruct((2, N, IN_CH // 2), jnp.bfloat16),
            jax.ShapeDtypeStruct((2, N, 16), jnp.float32),
            jax.ShapeDtypeStruct((2, N, 16), jnp.float32),
            jax.ShapeDtypeStruct((1, 16), jnp.float32),
            jax.ShapeDtypeStruct((1, 16), jnp.float32),
            jax.ShapeDtypeStruct((2, 16), jnp.float32),
        ],
    )(x, W1, av1, adv1)


def _tc2(acc1, den1, b1, W2, as2, ad2):
    return pl.pallas_call(
        _tc2_body,
        grid=(_G,),
        in_specs=[
            pl.BlockSpec((2, _BM, IN_CH // 2), lambda i: (0, i, 0)),
            pl.BlockSpec((2, _BM, 16), lambda i: (0, i, 0)),
            pl.BlockSpec((1, IN_CH), lambda i: (0, 0)),
            pl.BlockSpec((IN_CH, OUT_CH), lambda i: (0, 0)),
            pl.BlockSpec((OUT_CH, 1), lambda i: (0, 0)),
            pl.BlockSpec((OUT_CH, 1), lambda i: (0, 0)),
        ],
        out_specs=[
            pl.BlockSpec((2, _BM, OUT_CH // 2), lambda i: (0, i, 0)),
            pl.BlockSpec((2, _BM, 16), lambda i: (0, i, 0)),
            pl.BlockSpec((2, _BM, 16), lambda i: (0, i, 0)),
            pl.BlockSpec((1, 16), lambda i: (0, 0)),
            pl.BlockSpec((1, 16), lambda i: (0, 0)),
            pl.BlockSpec((2, 16), lambda i: (0, 0)),
        ],
        out_shape=[
            jax.ShapeDtypeStruct((2, N, OUT_CH // 2), jnp.bfloat16),
            jax.ShapeDtypeStruct((2, N, 16), jnp.float32),
            jax.ShapeDtypeStruct((2, N, 16), jnp.float32),
            jax.ShapeDtypeStruct((1, 16), jnp.float32),
            jax.ShapeDtypeStruct((1, 16), jnp.float32),
            jax.ShapeDtypeStruct((2, 16), jnp.float32),
        ],
    )(acc1, den1, b1, W2, as2, ad2)


def _tc3(acc2, den2, b2):
    return pl.pallas_call(
        _tc3_body,
        grid=(_G,),
        in_specs=[
            pl.BlockSpec((2, _BM, OUT_CH // 2), lambda i: (0, i, 0)),
            pl.BlockSpec((2, _BM, 16), lambda i: (0, i, 0)),
            pl.BlockSpec((1, OUT_CH), lambda i: (0, 0)),
        ],
        out_specs=pl.BlockSpec((_BM, OUT_CH), lambda i: (i, 0)),
        out_shape=jax.ShapeDtypeStruct((N, OUT_CH), jnp.float32),
    )(acc2, den2, b2)


def _sc_edge_pass(h2s, Q2s, R2s, C2s, src2d, dst2d, F):
    FH = F // 2
    ngrp = FH // 16

    @functools.partial(
        pl.kernel,
        out_type=[
            jax.ShapeDtypeStruct((2, N, FH), jnp.float32),
            jax.ShapeDtypeStruct((2, N, 16), jnp.float32),
        ],
        mesh=_sc_mesh(),
        compiler_params=pltpu.CompilerParams(use_tc_tiling_on_sc=False,
                                             needs_layout_passes=False),
        scratch_types=[
            pltpu.VMEM((NCPT, CH), jnp.int32),
            [pltpu.VMEM((CH,), jnp.int32)] * 3,
            [pltpu.VMEM((CH, 16), jnp.float32)] * 3,
            [pltpu.VMEM((CH, 16), jnp.float32)] * 3,
            [pltpu.VMEM((CH, FH), jnp.bfloat16)] * 3,
            [pltpu.VMEM((CH, FH), jnp.float32)] * 3,
            [pltpu.VMEM((CH, 16), jnp.float32)] * 3,
            pltpu.VMEM((1, 16), jnp.float32),
            pltpu.VMEM_SHARED((NROWS, FH), jnp.float32),
            pltpu.VMEM_SHARED((NROWS, 16), jnp.float32),
            [pltpu.SemaphoreType.DMA] * 3,
            [pltpu.SemaphoreType.DMA] * 3,
            [pltpu.SemaphoreType.DMA] * 3,
            [pltpu.SemaphoreType.DMA] * 3,
        ],
    )
    def k(h_hbm, q_hbm, r_hbm, c_hbm, s_hbm, d_hbm, acc_out, den_out,
          dslab, SB, QS, RD, HS, HM, EB, cvec, acc_sp, den_sp, IS, GS, SS, DS):
        cid = lax.axis_index("c")
        sid = lax.axis_index("s")

        hs0, eb0 = HM[0], EB[0]

        @pl.loop(0, CH)
        def _(r2):
            for j in range(ngrp):
                hs0[r2, pl.ds(j * 16, 16)] = jnp.zeros((16,), jnp.float32)
            eb0[r2, :] = jnp.zeros((16,), jnp.float32)

        for j in range(5):
            i = j * 16 + sid

            @pl.when(i < NFULL)
            def _():
                pltpu.sync_copy(hs0, acc_sp.at[pl.ds(i * CH, CH)])
                pltpu.sync_copy(eb0, den_sp.at[pl.ds(i * CH, CH)])

        @pl.when(sid == 15)
        def _():
            pltpu.sync_copy(hs0.at[pl.ds(0, NTAIL)],
                            acc_sp.at[pl.ds(NFULL * CH, NTAIL)])
            pltpu.sync_copy(eb0.at[pl.ds(0, NTAIL)],
                            den_sp.at[pl.ds(NFULL * CH, NTAIL)])

        pltpu.sync_copy(c_hbm.at[pl.ds(cid, 1)], cvec)
        pltpu.sync_copy(d_hbm.at[pl.ds(sid * NCPT, NCPT)], dslab)
        plsc.subcore_barrier()
        cv = cvec[0, :]
        base = sid * NCPT

        def fire_sidx(c, b):
            pltpu.async_copy(s_hbm.at[base + c], SB[b], IS[b])

        def wait_sidx(b):
            pltpu.make_async_copy(s_hbm.at[base], SB[b], IS[b]).wait()

        def fire_g(c, b):
            pltpu.async_copy(q_hbm.at[cid].at[SB[b]], QS[b], GS[b])
            pltpu.async_copy(r_hbm.at[cid].at[dslab.at[c]], RD[b], GS[b])
            pltpu.async_copy(h_hbm.at[cid].at[SB[b]], HS[b], GS[b])

        def wait_g(b):
            pltpu.make_async_copy(q_hbm.at[cid].at[SB[b]], QS[b], GS[b]).wait()
            pltpu.make_async_copy(r_hbm.at[cid].at[SB[b]], RD[b], GS[b]).wait()
            pltpu.make_async_copy(h_hbm.at[cid].at[SB[b]], HS[b], GS[b]).wait()

        def wait_s(b, c):
            @pl.when((c % 2) == cid)
            def _():
                pltpu.make_async_copy(EB[b], den_sp.at[dslab.at[0]],
                                      DS[b]).wait()

            pltpu.make_async_copy(HM[b], acc_sp.at[dslab.at[0]], SS[b]).wait()

        def compute_scatter(c, b):
            qs_b, rd_b, hs_b, hm_b, eb_b = QS[b], RD[b], HS[b], HM[b], EB[b]

            @plsc.parallel_loop(0, CH, unroll=4)
            def _(e):
                a = qs_b[e, :] + rd_b[e, :]
                al = jnp.maximum(a, 0.2 * a)
                exv = jnp.exp(al - cv)
                eb_b[e, :] = exv
                for g2 in range(FH // 32):
                    blk = hs_b[e, pl.ds(g2 * 32, 32)]
                    va, vb = plsc.unpack(blk, format=plsc.PackFormat.INTERLEAVED)
                    spa = jnp.full((16,), exv[2 * g2], jnp.float32)
                    spb = jnp.full((16,), exv[2 * g2 + 1], jnp.float32)
                    hm_b[e, pl.ds(g2 * 32, 16)] = va * spa
                    hm_b[e, pl.ds(g2 * 32 + 16, 16)] = vb * spb

            @pl.when((c % 2) == cid)
            def _():
                pltpu.async_copy(eb_b, den_sp.at[dslab.at[c]], DS[b], add=True)

            pltpu.async_copy(hm_b, acc_sp.at[dslab.at[c]], SS[b], add=True)

        def substep(i, b, do_ws, do_fi, do_fg):
            if do_ws:
                wait_s((b + 1) % 3, i - 2)
            if do_fi:
                fire_sidx(i + 2, (b + 2) % 3)
            if do_fg:
                wait_sidx((b + 1) % 3)
                fire_g(i + 1, (b + 1) % 3)
            wait_g(b)
            compute_scatter(i, b)

        fire_sidx(0, 0)
        fire_sidx(1, 1)
        wait_sidx(0)
        fire_g(0, 0)
        substep(0, 0, False, True, True)
        substep(1, 1, False, True, True)
        substep(2, 2, True, True, True)
        substep(3, 0, True, True, True)

        @pl.loop(0, 51)
        def _(m):
            i0 = 4 + m * 3
            substep(i0, 1, True, True, True)
            substep(i0 + 1, 2, True, True, True)
            substep(i0 + 2, 0, True, True, True)

        substep(NCPT - 2, 1, True, False, True)
        substep(NCPT - 1, 2, False, False, False)
        wait_s(0, NCPT - 3)
        wait_s(1, NCPT - 2)
        wait_s(2, NCPT - 1)

        plsc.subcore_barrier()

        for j in range(5):
            i = j * 16 + sid

            @pl.when(i < NFULL)
            def _():
                pltpu.sync_copy(acc_sp.at[pl.ds(i * CH, CH)],
                                acc_out.at[cid, pl.ds(i * CH, CH)])
                pltpu.sync_copy(den_sp.at[pl.ds(i * CH, CH)],
                                den_out.at[cid, pl.ds(i * CH, CH)])

        @pl.when(sid == 15)
        def _():
            pltpu.sync_copy(acc_sp.at[pl.ds(NFULL * CH, NTAIL)],
                            acc_out.at[cid, pl.ds(NFULL * CH, NTAIL)])
            pltpu.sync_copy(den_sp.at[pl.ds(NFULL * CH, NTAIL)],
                            den_out.at[cid, pl.ds(NFULL * CH, NTAIL)])

    return k(h2s, Q2s, R2s, C2s, src2d, dst2d)


def kernel(x, edge_index, W1, att_src1, att_dst1, b1, W2, att_src2, att_dst2, b2):
    src = edge_index[0].astype(jnp.int32)
    dst = edge_index[1].astype(jnp.int32)
    pad = E_PAD - E
    src2d = jnp.concatenate([src, jnp.zeros((pad,), jnp.int32)]).reshape(-1, CH)
    dst2d = jnp.concatenate([dst, jnp.full((pad,), N, jnp.int32)]).reshape(-1, CH)

    h1, Q1s, R1s, _, _, C1s = _tc1(x, W1, att_src1.reshape(1, IN_CH),
                                   att_dst1.reshape(1, IN_CH))
    acc1, den1 = _sc_edge_pass(h1, Q1s, R1s, C1s, src2d, dst2d, IN_CH)

    h2, S2s, D2s, _, _, C2s = _tc2(acc1, den1, b1.reshape(1, IN_CH), W2,
                                   att_src2.reshape(OUT_CH, 1),
                                   att_dst2.reshape(OUT_CH, 1))
    acc2, den2 = _sc_edge_pass(h2, S2s, D2s, C2s, src2d, dst2d, OUT_CH)

    return _tc3(acc2, den2, b2.reshape(1, OUT_CH))


# speedup vs baseline: 1.8533x; 1.0015x over previous
import functools

import jax
import jax.numpy as jnp
from jax import lax
from jax.experimental import pallas as pl
from jax.experimental.pallas import tpu as pltpu
from jax.experimental.pallas import tpu_sc as plsc

N = 10000
E = 320000
IN_CH = 128
HID = 16
HEADS = 8
OUT_CH = 64

_HIGH = jax.lax.Precision.HIGHEST
_BM = 1000
_G = N // _BM
CH = 128
NFULL = N // CH
NTAIL = N - NFULL * CH
NCPT = 159
E_PAD = 16 * NCPT * CH
NROWS = N + 8


def _sc_mesh():
    return plsc.VectorSubcoreMesh(core_axis_name="c", subcore_axis_name="s")


def _dot(a, b):
    return jnp.dot(a, b, precision=_HIGH, preferred_element_type=jnp.float32)


def _roll4(v):
    return jnp.concatenate([v[:, 4:], v[:, :4]], axis=1)


def _pack_perm(w):
    J = jax.lax.broadcasted_iota(jnp.int32, (w, w), 1)
    T = jax.lax.broadcasted_iota(jnp.int32, (w, w), 0)
    tj = 32 * (J // 32) + (J % 32) // 2 + 16 * (J % 2)
    return jnp.where(T == tj, 1.0, 0.0).astype(jnp.float32)


def _onehot_heads():
    lane = jax.lax.broadcasted_iota(jnp.int32, (IN_CH, 16), 1) % 8
    grp = jax.lax.broadcasted_iota(jnp.int32, (IN_CH, 16), 0) // HID
    return jnp.where(grp == lane, 1.0, 0.0).astype(jnp.float32)


def _tc1_body(x_ref, w_ref, as_ref, ad_ref, h_ref, q_ref, r_ref, qm_ref,
              rm_ref, c_ref):
    i = pl.program_id(0)
    h = _dot(x_ref[...], w_ref[...])
    perm = _pack_perm(IN_CH // 2)
    h_ref[0] = _dot(h[:, :IN_CH // 2], perm).astype(jnp.bfloat16)
    h_ref[1] = _dot(h[:, IN_CH // 2:], perm).astype(jnp.bfloat16)
    oneh = _onehot_heads()
    q = _dot(h * as_ref[...], oneh)
    r = _dot(h * ad_ref[...], oneh)
    q_ref[0] = q
    q_ref[1] = _roll4(q)
    r_ref[0] = r
    r_ref[1] = _roll4(r)
    qm = jnp.max(q, axis=0, keepdims=True)
    rm = jnp.max(r, axis=0, keepdims=True)

    @pl.when(i == 0)
    def _():
        qm_ref[...] = qm
        rm_ref[...] = rm

    @pl.when(i > 0)
    def _():
        qm_ref[...] = jnp.maximum(qm_ref[...], qm)
        rm_ref[...] = jnp.maximum(rm_ref[...], rm)

    @pl.when(i == _G - 1)
    def _():
        cs = qm_ref[...] + rm_ref[...]
        c = jnp.maximum(cs, 0.2 * cs)
        c_ref[...] = jnp.concatenate([c, _roll4(c)], axis=0)


def _tc2_body(acc_ref, den_ref, b1_ref, w2_ref, as_ref, ad_ref,
              h2_ref, s_ref, d_ref, sm_ref, dm_ref, c_ref):
    i = pl.program_id(0)
    acc = jnp.concatenate([acc_ref[0], acc_ref[1]], axis=1)
    den8 = den_ref[0][:, :8] + den_ref[1][:, :8]
    exp8 = jnp.where(
        jax.lax.broadcasted_iota(jnp.int32, (8, IN_CH), 0)
        == jax.lax.broadcasted_iota(jnp.int32, (8, IN_CH), 1) // HID,
        1.0, 0.0).astype(jnp.float32)
    den128 = _dot(den8, exp8)
    h1 = acc / (den128 + 1e-16) + b1_ref[...]
    h2 = _dot(h1, w2_ref[...])
    perm = _pack_perm(OUT_CH // 2)
    h2_ref[0] = _dot(h2[:, :OUT_CH // 2], perm).astype(jnp.bfloat16)
    h2_ref[1] = _dot(h2[:, OUT_CH // 2:], perm).astype(jnp.bfloat16)
    s = jnp.broadcast_to(_dot(h2, as_ref[...]), (h2.shape[0], 16))
    d = jnp.broadcast_to(_dot(h2, ad_ref[...]), (h2.shape[0], 16))
    s_ref[0] = s
    s_ref[1] = s
    d_ref[0] = d
    d_ref[1] = d
    sm = jnp.max(s, axis=0, keepdims=True)
    dm = jnp.max(d, axis=0, keepdims=True)

    @pl.when(i == 0)
    def _():
        sm_ref[...] = sm
        dm_ref[...] = dm

    @pl.when(i > 0)
    def _():
        sm_ref[...] = jnp.maximum(sm_ref[...], sm)
        dm_ref[...] = jnp.maximum(dm_ref[...], dm)

    @pl.when(i == _G - 1)
    def _():
        cs = sm_ref[...] + dm_ref[...]
        c = jnp.maximum(cs, 0.2 * cs)
        c_ref[...] = jnp.concatenate([c, c], axis=0)


def _tc3_body(acc_ref, den_ref, b2_ref, out_ref):
    acc = jnp.concatenate([acc_ref[0], acc_ref[1]], axis=1)
    den = den_ref[0][:, 0:1] + den_ref[1][:, 0:1]
    o = acc / (den + 1e-16) + b2_ref[...]
    m = jnp.max(o, axis=1, keepdims=True)
    z = o - m
    lse = jnp.log(jnp.sum(jnp.exp(z), axis=1, keepdims=True))
    out_ref[...] = z - lse


def _tc1(x, W1, av1, adv1):
    return pl.pallas_call(
        _tc1_body,
        grid=(_G,),
        in_specs=[
            pl.BlockSpec((_BM, IN_CH), lambda i: (i, 0)),
            pl.BlockSpec((IN_CH, IN_CH), lambda i: (0, 0)),
            pl.BlockSpec((1, IN_CH), lambda i: (0, 0)),
            pl.BlockSpec((1, IN_CH), lambda i: (0, 0)),
        ],
        out_specs=[
            pl.BlockSpec((2, _BM, IN_CH // 2), lambda i: (0, i, 0)),
            pl.BlockSpec((2, _BM, 16), lambda i: (0, i, 0)),
            pl.BlockSpec((2, _BM, 16), lambda i: (0, i, 0)),
            pl.BlockSpec((1, 16), lambda i: (0, 0)),
            pl.BlockSpec((1, 16), lambda i: (0, 0)),
            pl.BlockSpec((2, 16), lambda i: (0, 0)),
        ],
        out_shape=[
            jax.ShapeDtypeStruct((2, N, IN_CH // 2), jnp.bfloat16),
            jax.ShapeDtypeStruct((2, N, 16), jnp.float32),
            jax.ShapeDtypeStruct((2, N, 16), jnp.float32),
            jax.ShapeDtypeStruct((1, 16), jnp.float32),
            jax.ShapeDtypeStruct((1, 16), jnp.float32),
            jax.ShapeDtypeStruct((2, 16), jnp.float32),
        ],
    )(x, W1, av1, adv1)


def _tc2(acc1, den1, b1, W2, as2, ad2):
    return pl.pallas_call(
        _tc2_body,
        grid=(_G,),
        in_specs=[
            pl.BlockSpec((2, _BM, IN_CH // 2), lambda i: (0, i, 0)),
            pl.BlockSpec((2, _BM, 16), lambda i: (0, i, 0)),
            pl.BlockSpec((1, IN_CH), lambda i: (0, 0)),
            pl.BlockSpec((IN_CH, OUT_CH), lambda i: (0, 0)),
            pl.BlockSpec((OUT_CH, 1), lambda i: (0, 0)),
            pl.BlockSpec((OUT_CH, 1), lambda i: (0, 0)),
        ],
        out_specs=[
            pl.BlockSpec((2, _BM, OUT_CH // 2), lambda i: (0, i, 0)),
            pl.BlockSpec((2, _BM, 16), lambda i: (0, i, 0)),
            pl.BlockSpec((2, _BM, 16), lambda i: (0, i, 0)),
            pl.BlockSpec((1, 16), lambda i: (0, 0)),
            pl.BlockSpec((1, 16), lambda i: (0, 0)),
            pl.BlockSpec((2, 16), lambda i: (0, 0)),
        ],
        out_shape=[
            jax.ShapeDtypeStruct((2, N, OUT_CH // 2), jnp.bfloat16),
            jax.ShapeDtypeStruct((2, N, 16), jnp.float32),
            jax.ShapeDtypeStruct((2, N, 16), jnp.float32),
            jax.ShapeDtypeStruct((1, 16), jnp.float32),
            jax.ShapeDtypeStruct((1, 16), jnp.float32),
            jax.ShapeDtypeStruct((2, 16), jnp.float32),
        ],
    )(acc1, den1, b1, W2, as2, ad2)


def _tc3(acc2, den2, b2):
    return pl.pallas_call(
        _tc3_body,
        grid=(_G,),
        in_specs=[
            pl.BlockSpec((2, _BM, OUT_CH // 2), lambda i: (0, i, 0)),
            pl.BlockSpec((2, _BM, 16), lambda i: (0, i, 0)),
            pl.BlockSpec((1, OUT_CH), lambda i: (0, 0)),
        ],
        out_specs=pl.BlockSpec((_BM, OUT_CH), lambda i: (i, 0)),
        out_shape=jax.ShapeDtypeStruct((N, OUT_CH), jnp.float32),
    )(acc2, den2, b2)


def _sc_edge_pass(h2s, Q2s, R2s, C2s, src2d, dst2d, F):
    FH = F // 2
    ngrp = FH // 16

    @functools.partial(
        pl.kernel,
        out_type=[
            jax.ShapeDtypeStruct((2, N, FH), jnp.float32),
            jax.ShapeDtypeStruct((2, N, 16), jnp.float32),
        ],
        mesh=_sc_mesh(),
        compiler_params=pltpu.CompilerParams(use_tc_tiling_on_sc=False,
                                             needs_layout_passes=False),
        scratch_types=[
            pltpu.VMEM((NCPT, CH), jnp.int32),
            [pltpu.VMEM((CH,), jnp.int32)] * 3,
            [pltpu.VMEM((CH, 16), jnp.float32)] * 3,
            [pltpu.VMEM((CH, 16), jnp.float32)] * 3,
            [pltpu.VMEM((CH, FH), jnp.bfloat16)] * 3,
            [pltpu.VMEM((CH, FH), jnp.float32)] * 3,
            [pltpu.VMEM((CH, 16), jnp.float32)] * 3,
            pltpu.VMEM((1, 16), jnp.float32),
            pltpu.VMEM_SHARED((NROWS, FH), jnp.float32),
            pltpu.VMEM_SHARED((NROWS, 16), jnp.float32),
            [pltpu.SemaphoreType.DMA] * 3,
            [pltpu.SemaphoreType.DMA] * 3,
            [pltpu.SemaphoreType.DMA] * 3,
            [pltpu.SemaphoreType.DMA] * 3,
        ],
    )
    def k(h_hbm, q_hbm, r_hbm, c_hbm, s_hbm, d_hbm, acc_out, den_out,
          dslab, SB, QS, RD, HS, HM, EB, cvec, acc_sp, den_sp, IS, GS, SS, DS):
        cid = lax.axis_index("c")
        sid = lax.axis_index("s")

        hs0, eb0 = HM[0], EB[0]

        @pl.loop(0, CH)
        def _(r2):
            for j in range(ngrp):
                hs0[r2, pl.ds(j * 16, 16)] = jnp.zeros((16,), jnp.float32)
            eb0[r2, :] = jnp.zeros((16,), jnp.float32)

        for j in range(5):
            i = j * 16 + sid

            @pl.when(i < NFULL)
            def _():
                pltpu.sync_copy(hs0, acc_sp.at[pl.ds(i * CH, CH)])
                pltpu.sync_copy(eb0, den_sp.at[pl.ds(i * CH, CH)])

        @pl.when(sid == 15)
        def _():
            pltpu.sync_copy(hs0.at[pl.ds(0, NTAIL)],
                            acc_sp.at[pl.ds(NFULL * CH, NTAIL)])
            pltpu.sync_copy(eb0.at[pl.ds(0, NTAIL)],
                            den_sp.at[pl.ds(NFULL * CH, NTAIL)])

        pltpu.sync_copy(c_hbm.at[pl.ds(cid, 1)], cvec)
        pltpu.sync_copy(d_hbm.at[pl.ds(sid * NCPT, NCPT)], dslab)
        plsc.subcore_barrier()
        cv = cvec[0, :]
        base = sid * NCPT

        def fire_sidx(c, b):
            pltpu.async_copy(s_hbm.at[base + c], SB[b], IS[b])

        def wait_sidx(b):
            pltpu.make_async_copy(s_hbm.at[base], SB[b], IS[b]).wait()

        def fire_g(c, b):
            pltpu.async_copy(q_hbm.at[cid].at[SB[b]], QS[b], GS[b])
            pltpu.async_copy(r_hbm.at[cid].at[dslab.at[c]], RD[b], GS[b])
            pltpu.async_copy(h_hbm.at[cid].at[SB[b]], HS[b], GS[b])

        def wait_g(b):
            pltpu.make_async_copy(q_hbm.at[cid].at[SB[b]], QS[b], GS[b]).wait()
            pltpu.make_async_copy(r_hbm.at[cid].at[SB[b]], RD[b], GS[b]).wait()
            pltpu.make_async_copy(h_hbm.at[cid].at[SB[b]], HS[b], GS[b]).wait()

        def wait_s(b, c):
            @pl.when((c % 2) == cid)
            def _():
                pltpu.make_async_copy(EB[b], den_sp.at[dslab.at[0]],
                                      DS[b]).wait()

            pltpu.make_async_copy(HM[b], acc_sp.at[dslab.at[0]], SS[b]).wait()

        def compute_scatter(c, b):
            qs_b, rd_b, hs_b, hm_b, eb_b = QS[b], RD[b], HS[b], HM[b], EB[b]

            @plsc.parallel_loop(0, CH, unroll=8)
            def _(e):
                a = qs_b[e, :] + rd_b[e, :]
                al = jnp.maximum(a, 0.2 * a)
                exv = jnp.exp(al - cv)
                eb_b[e, :] = exv
                for g2 in range(FH // 32):
                    blk = hs_b[e, pl.ds(g2 * 32, 32)]
                    va, vb = plsc.unpack(blk, format=plsc.PackFormat.INTERLEAVED)
                    spa = jnp.full((16,), exv[2 * g2], jnp.float32)
                    spb = jnp.full((16,), exv[2 * g2 + 1], jnp.float32)
                    hm_b[e, pl.ds(g2 * 32, 16)] = va * spa
                    hm_b[e, pl.ds(g2 * 32 + 16, 16)] = vb * spb

            @pl.when((c % 2) == cid)
            def _():
                pltpu.async_copy(eb_b, den_sp.at[dslab.at[c]], DS[b], add=True)

            pltpu.async_copy(hm_b, acc_sp.at[dslab.at[c]], SS[b], add=True)

        def substep(i, b, do_ws, do_fi, do_fg):
            if do_ws:
                wait_s((b + 1) % 3, i - 2)
            if do_fi:
                fire_sidx(i + 2, (b + 2) % 3)
            if do_fg:
                wait_sidx((b + 1) % 3)
                fire_g(i + 1, (b + 1) % 3)
            wait_g(b)
            compute_scatter(i, b)

        fire_sidx(0, 0)
        fire_sidx(1, 1)
        wait_sidx(0)
        fire_g(0, 0)
        substep(0, 0, False, True, True)
        substep(1, 1, False, True, True)
        substep(2, 2, True, True, True)
        substep(3, 0, True, True, True)

        @pl.loop(0, 51)
        def _(m):
            i0 = 4 + m * 3
            substep(i0, 1, True, True, True)
            substep(i0 + 1, 2, True, True, True)
            substep(i0 + 2, 0, True, True, True)

        substep(NCPT - 2, 1, True, False, True)
        substep(NCPT - 1, 2, False, False, False)
        wait_s(0, NCPT - 3)
        wait_s(1, NCPT - 2)
        wait_s(2, NCPT - 1)

        plsc.subcore_barrier()

        for j in range(5):
            i = j * 16 + sid

            @pl.when(i < NFULL)
            def _():
                pltpu.sync_copy(acc_sp.at[pl.ds(i * CH, CH)],
                                acc_out.at[cid, pl.ds(i * CH, CH)])
                pltpu.sync_copy(den_sp.at[pl.ds(i * CH, CH)],
                                den_out.at[cid, pl.ds(i * CH, CH)])

        @pl.when(sid == 15)
        def _():
            pltpu.sync_copy(acc_sp.at[pl.ds(NFULL * CH, NTAIL)],
                            acc_out.at[cid, pl.ds(NFULL * CH, NTAIL)])
            pltpu.sync_copy(den_sp.at[pl.ds(NFULL * CH, NTAIL)],
                            den_out.at[cid, pl.ds(NFULL * CH, NTAIL)])

    return k(h2s, Q2s, R2s, C2s, src2d, dst2d)


def kernel(x, edge_index, W1, att_src1, att_dst1, b1, W2, att_src2, att_dst2, b2):
    src = edge_index[0].astype(jnp.int32)
    dst = edge_index[1].astype(jnp.int32)
    pad = E_PAD - E
    src2d = jnp.concatenate([src, jnp.zeros((pad,), jnp.int32)]).reshape(-1, CH)
    dst2d = jnp.concatenate([dst, jnp.full((pad,), N, jnp.int32)]).reshape(-1, CH)

    h1, Q1s, R1s, _, _, C1s = _tc1(x, W1, att_src1.reshape(1, IN_CH),
                                   att_dst1.reshape(1, IN_CH))
    acc1, den1 = _sc_edge_pass(h1, Q1s, R1s, C1s, src2d, dst2d, IN_CH)

    h2, S2s, D2s, _, _, C2s = _tc2(acc1, den1, b1.reshape(1, IN_CH), W2,
                                   att_src2.reshape(OUT_CH, 1),
                                   att_dst2.reshape(OUT_CH, 1))
    acc2, den2 = _sc_edge_pass(h2, S2s, D2s, C2s, src2d, dst2d, OUT_CH)

    return _tc3(acc2, den2, b2.reshape(1, OUT_CH))


# speedup vs baseline: 1.8601x; 1.0036x over previous
import functools

import jax
import jax.numpy as jnp
from jax import lax
from jax.experimental import pallas as pl
from jax.experimental.pallas import tpu as pltpu
from jax.experimental.pallas import tpu_sc as plsc

N = 10000
E = 320000
IN_CH = 128
HID = 16
HEADS = 8
OUT_CH = 64

_HIGH = jax.lax.Precision.HIGHEST
_BM = 1000
_G = N // _BM
CH = 128
NFULL = N // CH
NTAIL = N - NFULL * CH
NCPT = 159
E_PAD = 16 * NCPT * CH
NROWS = N + 8


def _sc_mesh():
    return plsc.VectorSubcoreMesh(core_axis_name="c", subcore_axis_name="s")


def _dot(a, b):
    return jnp.dot(a, b, precision=_HIGH, preferred_element_type=jnp.float32)


def _roll4(v):
    return jnp.concatenate([v[:, 4:], v[:, :4]], axis=1)


def _pack_perm(w):
    J = jax.lax.broadcasted_iota(jnp.int32, (w, w), 1)
    T = jax.lax.broadcasted_iota(jnp.int32, (w, w), 0)
    tj = 32 * (J // 32) + (J % 32) // 2 + 16 * (J % 2)
    return jnp.where(T == tj, 1.0, 0.0).astype(jnp.float32)


def _onehot_heads():
    lane = jax.lax.broadcasted_iota(jnp.int32, (IN_CH, 16), 1) % 8
    grp = jax.lax.broadcasted_iota(jnp.int32, (IN_CH, 16), 0) // HID
    return jnp.where(grp == lane, 1.0, 0.0).astype(jnp.float32)


def _tc1_body(x_ref, w_ref, as_ref, ad_ref, h_ref, q_ref, r_ref, qm_ref,
              rm_ref, c_ref):
    i = pl.program_id(0)
    h = _dot(x_ref[...], w_ref[...])
    perm = _pack_perm(IN_CH // 2)
    h_ref[0] = _dot(h[:, :IN_CH // 2], perm).astype(jnp.bfloat16)
    h_ref[1] = _dot(h[:, IN_CH // 2:], perm).astype(jnp.bfloat16)
    oneh = _onehot_heads()
    q = _dot(h * as_ref[...], oneh)
    r = _dot(h * ad_ref[...], oneh)
    q_ref[0] = q
    q_ref[1] = _roll4(q)
    r_ref[0] = r
    r_ref[1] = _roll4(r)
    qm = jnp.max(q, axis=0, keepdims=True)
    rm = jnp.max(r, axis=0, keepdims=True)

    @pl.when(i == 0)
    def _():
        qm_ref[...] = qm
        rm_ref[...] = rm

    @pl.when(i > 0)
    def _():
        qm_ref[...] = jnp.maximum(qm_ref[...], qm)
        rm_ref[...] = jnp.maximum(rm_ref[...], rm)

    @pl.when(i == _G - 1)
    def _():
        cs = qm_ref[...] + rm_ref[...]
        c = jnp.maximum(cs, 0.2 * cs)
        c_ref[...] = jnp.concatenate([c, _roll4(c)], axis=0)


def _tc2_body(acc_ref, den_ref, b1_ref, w2_ref, as_ref, ad_ref,
              h2_ref, s_ref, d_ref, sm_ref, dm_ref, c_ref):
    i = pl.program_id(0)
    acc = jnp.concatenate([acc_ref[0], acc_ref[1]], axis=1)
    den8 = den_ref[0][:, :8] + den_ref[1][:, :8]
    exp8 = jnp.where(
        jax.lax.broadcasted_iota(jnp.int32, (8, IN_CH), 0)
        == jax.lax.broadcasted_iota(jnp.int32, (8, IN_CH), 1) // HID,
        1.0, 0.0).astype(jnp.float32)
    den128 = _dot(den8, exp8)
    h1 = acc / (den128 + 1e-16) + b1_ref[...]
    h2 = _dot(h1, w2_ref[...])
    perm = _pack_perm(OUT_CH // 2)
    h2_ref[0] = _dot(h2[:, :OUT_CH // 2], perm).astype(jnp.bfloat16)
    h2_ref[1] = _dot(h2[:, OUT_CH // 2:], perm).astype(jnp.bfloat16)
    s = jnp.broadcast_to(_dot(h2, as_ref[...]), (h2.shape[0], 16))
    d = jnp.broadcast_to(_dot(h2, ad_ref[...]), (h2.shape[0], 16))
    s_ref[0] = s
    s_ref[1] = s
    d_ref[0] = d
    d_ref[1] = d
    sm = jnp.max(s, axis=0, keepdims=True)
    dm = jnp.max(d, axis=0, keepdims=True)

    @pl.when(i == 0)
    def _():
        sm_ref[...] = sm
        dm_ref[...] = dm

    @pl.when(i > 0)
    def _():
        sm_ref[...] = jnp.maximum(sm_ref[...], sm)
        dm_ref[...] = jnp.maximum(dm_ref[...], dm)

    @pl.when(i == _G - 1)
    def _():
        cs = sm_ref[...] + dm_ref[...]
        c = jnp.maximum(cs, 0.2 * cs)
        c_ref[...] = jnp.concatenate([c, c], axis=0)


def _tc3_body(acc_ref, den_ref, b2_ref, out_ref):
    acc = jnp.concatenate([acc_ref[0], acc_ref[1]], axis=1)
    den = den_ref[0][:, 0:1] + den_ref[1][:, 0:1]
    o = acc / (den + 1e-16) + b2_ref[...]
    m = jnp.max(o, axis=1, keepdims=True)
    z = o - m
    lse = jnp.log(jnp.sum(jnp.exp(z), axis=1, keepdims=True))
    out_ref[...] = z - lse


def _tc1(x, W1, av1, adv1):
    return pl.pallas_call(
        _tc1_body,
        grid=(_G,),
        in_specs=[
            pl.BlockSpec((_BM, IN_CH), lambda i: (i, 0)),
            pl.BlockSpec((IN_CH, IN_CH), lambda i: (0, 0)),
            pl.BlockSpec((1, IN_CH), lambda i: (0, 0)),
            pl.BlockSpec((1, IN_CH), lambda i: (0, 0)),
        ],
        out_specs=[
            pl.BlockSpec((2, _BM, IN_CH // 2), lambda i: (0, i, 0)),
            pl.BlockSpec((2, _BM, 16), lambda i: (0, i, 0)),
            pl.BlockSpec((2, _BM, 16), lambda i: (0, i, 0)),
            pl.BlockSpec((1, 16), lambda i: (0, 0)),
            pl.BlockSpec((1, 16), lambda i: (0, 0)),
            pl.BlockSpec((2, 16), lambda i: (0, 0)),
        ],
        out_shape=[
            jax.ShapeDtypeStruct((2, N, IN_CH // 2), jnp.bfloat16),
            jax.ShapeDtypeStruct((2, N, 16), jnp.float32),
            jax.ShapeDtypeStruct((2, N, 16), jnp.float32),
            jax.ShapeDtypeStruct((1, 16), jnp.float32),
            jax.ShapeDtypeStruct((1, 16), jnp.float32),
            jax.ShapeDtypeStruct((2, 16), jnp.float32),
        ],
    )(x, W1, av1, adv1)


def _tc2(acc1, den1, b1, W2, as2, ad2):
    return pl.pallas_call(
        _tc2_body,
        grid=(_G,),
        in_specs=[
            pl.BlockSpec((2, _BM, IN_CH // 2), lambda i: (0, i, 0)),
            pl.BlockSpec((2, _BM, 16), lambda i: (0, i, 0)),
            pl.BlockSpec((1, IN_CH), lambda i: (0, 0)),
            pl.BlockSpec((IN_CH, OUT_CH), lambda i: (0, 0)),
            pl.BlockSpec((OUT_CH, 1), lambda i: (0, 0)),
            pl.BlockSpec((OUT_CH, 1), lambda i: (0, 0)),
        ],
        out_specs=[
            pl.BlockSpec((2, _BM, OUT_CH // 2), lambda i: (0, i, 0)),
            pl.BlockSpec((2, _BM, 16), lambda i: (0, i, 0)),
            pl.BlockSpec((2, _BM, 16), lambda i: (0, i, 0)),
            pl.BlockSpec((1, 16), lambda i: (0, 0)),
            pl.BlockSpec((1, 16), lambda i: (0, 0)),
            pl.BlockSpec((2, 16), lambda i: (0, 0)),
        ],
        out_shape=[
            jax.ShapeDtypeStruct((2, N, OUT_CH // 2), jnp.bfloat16),
            jax.ShapeDtypeStruct((2, N, 16), jnp.float32),
            jax.ShapeDtypeStruct((2, N, 16), jnp.float32),
            jax.ShapeDtypeStruct((1, 16), jnp.float32),
            jax.ShapeDtypeStruct((1, 16), jnp.float32),
            jax.ShapeDtypeStruct((2, 16), jnp.float32),
        ],
    )(acc1, den1, b1, W2, as2, ad2)


def _tc3(acc2, den2, b2):
    return pl.pallas_call(
        _tc3_body,
        grid=(_G,),
        in_specs=[
            pl.BlockSpec((2, _BM, OUT_CH // 2), lambda i: (0, i, 0)),
            pl.BlockSpec((2, _BM, 16), lambda i: (0, i, 0)),
            pl.BlockSpec((1, OUT_CH), lambda i: (0, 0)),
        ],
        out_specs=pl.BlockSpec((_BM, OUT_CH), lambda i: (i, 0)),
        out_shape=jax.ShapeDtypeStruct((N, OUT_CH), jnp.float32),
    )(acc2, den2, b2)


def _sc_edge_pass(h2s, Q2s, R2s, C2s, src2d, dst2d, F):
    FH = F // 2
    ngrp = FH // 16

    @functools.partial(
        pl.kernel,
        out_type=[
            jax.ShapeDtypeStruct((2, N, FH), jnp.float32),
            jax.ShapeDtypeStruct((2, N, 16), jnp.float32),
        ],
        mesh=_sc_mesh(),
        compiler_params=pltpu.CompilerParams(use_tc_tiling_on_sc=False,
                                             needs_layout_passes=False),
        scratch_types=[
            pltpu.VMEM((NCPT, CH), jnp.int32),
            [pltpu.VMEM((CH,), jnp.int32)] * 3,
            [pltpu.VMEM((CH, 16), jnp.float32)] * 3,
            [pltpu.VMEM((CH, 16), jnp.float32)] * 3,
            [pltpu.VMEM((CH, FH), jnp.bfloat16)] * 3,
            [pltpu.VMEM((CH, FH), jnp.float32)] * 3,
            [pltpu.VMEM((CH, 16), jnp.float32)] * 3,
            pltpu.VMEM((1, 16), jnp.float32),
            pltpu.VMEM_SHARED((NROWS, FH), jnp.float32),
            pltpu.VMEM_SHARED((NROWS, 16), jnp.float32),
            [pltpu.SemaphoreType.DMA] * 3,
            [pltpu.SemaphoreType.DMA] * 3,
            [pltpu.SemaphoreType.DMA] * 3,
            [pltpu.SemaphoreType.DMA] * 3,
        ],
    )
    def k(h_hbm, q_hbm, r_hbm, c_hbm, s_hbm, d_hbm, acc_out, den_out,
          dslab, SB, QS, RD, HS, HM, EB, cvec, acc_sp, den_sp, IS, GS, SS, DS):
        cid = lax.axis_index("c")
        sid = lax.axis_index("s")

        hs0, eb0 = HM[0], EB[0]

        @pl.loop(0, CH)
        def _(r2):
            for j in range(ngrp):
                hs0[r2, pl.ds(j * 16, 16)] = jnp.zeros((16,), jnp.float32)
            eb0[r2, :] = jnp.zeros((16,), jnp.float32)

        for j in range(5):
            i = j * 16 + sid

            @pl.when(i < NFULL)
            def _():
                pltpu.sync_copy(hs0, acc_sp.at[pl.ds(i * CH, CH)])
                pltpu.sync_copy(eb0, den_sp.at[pl.ds(i * CH, CH)])

        @pl.when(sid == 15)
        def _():
            pltpu.sync_copy(hs0.at[pl.ds(0, NTAIL)],
                            acc_sp.at[pl.ds(NFULL * CH, NTAIL)])
            pltpu.sync_copy(eb0.at[pl.ds(0, NTAIL)],
                            den_sp.at[pl.ds(NFULL * CH, NTAIL)])

        pltpu.sync_copy(c_hbm.at[pl.ds(cid, 1)], cvec)
        pltpu.sync_copy(d_hbm.at[pl.ds(sid * NCPT, NCPT)], dslab)
        plsc.subcore_barrier()
        cv = cvec[0, :]
        base = sid * NCPT

        def fire_sidx(c, b):
            pltpu.async_copy(s_hbm.at[base + c], SB[b], IS[b])

        def wait_sidx(b):
            pltpu.make_async_copy(s_hbm.at[base], SB[b], IS[b]).wait()

        def fire_g(c, b):
            pltpu.async_copy(q_hbm.at[cid].at[SB[b]], QS[b], GS[b])
            pltpu.async_copy(r_hbm.at[cid].at[dslab.at[c]], RD[b], GS[b])
            pltpu.async_copy(h_hbm.at[cid].at[SB[b]], HS[b], GS[b])

        def wait_g(b):
            pltpu.make_async_copy(q_hbm.at[cid].at[SB[b]], QS[b], GS[b]).wait()
            pltpu.make_async_copy(r_hbm.at[cid].at[SB[b]], RD[b], GS[b]).wait()
            pltpu.make_async_copy(h_hbm.at[cid].at[SB[b]], HS[b], GS[b]).wait()

        def wait_s(b, c):
            @pl.when((c % 2) == cid)
            def _():
                pltpu.make_async_copy(EB[b], den_sp.at[dslab.at[0]],
                                      DS[b]).wait()

            pltpu.make_async_copy(HM[b], acc_sp.at[dslab.at[0]], SS[b]).wait()

        def compute_scatter(c, b):
            qs_b, rd_b, hs_b, hm_b, eb_b = QS[b], RD[b], HS[b], HM[b], EB[b]

            @plsc.parallel_loop(0, CH, unroll=4)
            def _(e):
                a = qs_b[e, :] + rd_b[e, :]
                al = jnp.maximum(a, 0.2 * a)
                exv = jnp.exp(al - cv)
                eb_b[e, :] = exv
                for g2 in range(FH // 32):
                    blk = hs_b[e, pl.ds(g2 * 32, 32)]
                    va, vb = plsc.unpack(blk, format=plsc.PackFormat.INTERLEAVED)
                    spa = jnp.full((16,), exv[2 * g2], jnp.float32)
                    spb = jnp.full((16,), exv[2 * g2 + 1], jnp.float32)
                    hm_b[e, pl.ds(g2 * 32, 16)] = va * spa
                    hm_b[e, pl.ds(g2 * 32 + 16, 16)] = vb * spb

            @pl.when((c % 2) == cid)
            def _():
                pltpu.async_copy(eb_b, den_sp.at[dslab.at[c]], DS[b], add=True)

            pltpu.async_copy(hm_b, acc_sp.at[dslab.at[c]], SS[b], add=True)

        def substep(i, b, do_ws, do_fi, do_fg):
            if do_ws:
                wait_s((b + 1) % 3, i - 2)
            if do_fi:
                fire_sidx(i + 2, (b + 2) % 3)
            if do_fg:
                wait_sidx((b + 1) % 3)
                fire_g(i + 1, (b + 1) % 3)
            wait_g(b)
            compute_scatter(i, b)

        fire_sidx(0, 0)
        fire_sidx(1, 1)
        wait_sidx(0)
        fire_g(0, 0)
        substep(0, 0, False, True, True)
        substep(1, 1, False, True, True)
        substep(2, 2, True, True, True)
        substep(3, 0, True, True, True)

        @pl.loop(0, 51)
        def _(m):
            i0 = 4 + m * 3
            substep(i0, 1, True, True, True)
            substep(i0 + 1, 2, True, True, True)
            substep(i0 + 2, 0, True, True, True)

        substep(NCPT - 2, 1, True, False, True)
        substep(NCPT - 1, 2, False, False, False)
        wait_s(0, NCPT - 3)
        wait_s(1, NCPT - 2)
        wait_s(2, NCPT - 1)

        plsc.subcore_barrier()

        for j in range(5):
            i = j * 16 + sid

            @pl.when(i < NFULL)
            def _():
                pltpu.sync_copy(acc_sp.at[pl.ds(i * CH, CH)],
                                acc_out.at[cid, pl.ds(i * CH, CH)])
                pltpu.sync_copy(den_sp.at[pl.ds(i * CH, CH)],
                                den_out.at[cid, pl.ds(i * CH, CH)])

        @pl.when(sid == 15)
        def _():
            pltpu.sync_copy(acc_sp.at[pl.ds(NFULL * CH, NTAIL)],
                            acc_out.at[cid, pl.ds(NFULL * CH, NTAIL)])
            pltpu.sync_copy(den_sp.at[pl.ds(NFULL * CH, NTAIL)],
                            den_out.at[cid, pl.ds(NFULL * CH, NTAIL)])

    return k(h2s, Q2s, R2s, C2s, src2d, dst2d)


def kernel(x, edge_index, W1, att_src1, att_dst1, b1, W2, att_src2, att_dst2, b2):
    src = edge_index[0].astype(jnp.int32)
    dst = edge_index[1].astype(jnp.int32)
    pad = E_PAD - E
    src2d = jnp.concatenate([src, jnp.zeros((pad,), jnp.int32)]).reshape(-1, CH)
    dst2d = jnp.concatenate([dst, jnp.full((pad,), N, jnp.int32)]).reshape(-1, CH)

    h1, Q1s, R1s, _, _, C1s = _tc1(x, W1, att_src1.reshape(1, IN_CH),
                                   att_dst1.reshape(1, IN_CH))
    acc1, den1 = _sc_edge_pass(h1, Q1s, R1s, C1s, src2d, dst2d, IN_CH)

    h2, S2s, D2s, _, _, C2s = _tc2(acc1, den1, b1.reshape(1, IN_CH), W2,
                                   att_src2.reshape(OUT_CH, 1),
                                   att_dst2.reshape(OUT_CH, 1))
    acc2, den2 = _sc_edge_pass(h2, S2s, D2s, C2s, src2d, dst2d, OUT_CH)

    return _tc3(acc2, den2, b2.reshape(1, OUT_CH))
